# baseline (speedup 1.0000x reference)
import functools

import jax
import jax.numpy as jnp
from jax import lax
from jax.experimental import pallas as pl
from jax.experimental.pallas import tpu as pltpu
import jax.experimental.pallas.tpu_sc as plsc

G = 5000
P = 10000
H = 128
B = 8
E = 320000
EPS = 1e-5

NC, NS = 2, 16
NW = NC * NS
EPW = E // NW
WIN = 80
NWIN = EPW // WIN
DEG_PAD = 10240
SLOTS = 16
CS_PAD = 16 * 10240

_f32 = jnp.float32
_i32 = jnp.int32


def _sc_body(src_h, dst_h, w_h, needed_h, deg_out, c_out,
             needed_v, src_b, dst_b, w_b, idx_b, val_b, zb, slotmap,
             deg_sh, c_sh):
    cid = lax.axis_index("c")
    sid = lax.axis_index("s")
    wid = cid * NS + sid

    def _z(i, _):
        zb[pl.ds(i * 16, 16)] = jnp.zeros((16,), _f32)
        return 0
    lax.fori_loop(0, (CS_PAD // NS) // 16, _z, 0)
    pltpu.sync_copy(zb, c_sh.at[pl.ds(sid * (CS_PAD // NS), CS_PAD // NS)])
    pltpu.sync_copy(zb.at[pl.ds(0, DEG_PAD // NS)],
                    deg_sh.at[pl.ds(sid * (DEG_PAD // NS), DEG_PAD // NS)])

    pltpu.sync_copy(needed_h, needed_v)
    pltpu.sync_copy(src_h.at[wid], src_b)
    pltpu.sync_copy(dst_h.at[wid], dst_b)
    pltpu.sync_copy(w_h.at[wid], w_b)

    def _zs(i, _):
        slotmap[pl.ds(i * 16, 16)] = jnp.zeros((16,), _i32)
        return 0
    lax.fori_loop(0, P // 16, _zs, 0)
    needed_vec = needed_v[...]
    repv = jnp.full((16,), SLOTS, _i32)
    for s in range(SLOTS):
        ns = needed_vec[s]
        repv = jnp.minimum(repv, jnp.where(needed_vec == ns, s, SLOTS))
    plsc.store_scatter(slotmap, [needed_vec], repv + 1)

    plsc.subcore_barrier()

    def _compute(j, _):
        for k in range(WIN // 16):
            off = k * 16
            srcv = src_b[j, pl.ds(off, 16)]
            dstv = dst_b[j, pl.ds(off, 16)]
            wv = w_b[j, pl.ds(off, 16)]
            slotv = plsc.load_gather(slotmap, [dstv])
            idx = srcv * SLOTS + jnp.maximum(slotv - 1, 0)
            val = jnp.where(slotv > 0, wv, jnp.zeros((16,), _f32))
            idx_b[j, pl.ds(off, 16)] = idx
            val_b[j, pl.ds(off, 16)] = val
        return 0
    lax.fori_loop(0, NWIN, _compute, 0)

    def _scatter(j, _):
        pltpu.sync_copy(w_b.at[j], deg_sh.at[dst_b.at[j]], add=True)
        pltpu.sync_copy(val_b.at[j], c_sh.at[idx_b.at[j]], add=True)
        return 0
    lax.fori_loop(0, NWIN, _scatter, 0)

    plsc.subcore_barrier()

    pltpu.sync_copy(deg_sh.at[pl.ds(sid * (DEG_PAD // NS), DEG_PAD // NS)],
                    deg_out.at[cid, 0, pl.ds(sid * (DEG_PAD // NS), DEG_PAD // NS)])
    pltpu.sync_copy(c_sh.at[pl.ds(sid * (CS_PAD // NS), CS_PAD // NS)],
                    c_out.at[cid, 0, pl.ds(sid * (CS_PAD // NS), CS_PAD // NS)])


def _sc_edges(src2, dst2, w2, needed):
    mesh = plsc.VectorSubcoreMesh(core_axis_name="c", subcore_axis_name="s",
                                  num_cores=NC, num_subcores=NS)
    kern = pl.kernel(
        _sc_body,
        out_type=(jax.ShapeDtypeStruct((NC, 1, DEG_PAD), _f32),
                  jax.ShapeDtypeStruct((NC, 1, CS_PAD), _f32)),
        mesh=mesh,
        scratch_types=dict(
            needed_v=pltpu.VMEM((16,), _i32),
            src_b=pltpu.VMEM((NWIN, WIN), _i32),
            dst_b=pltpu.VMEM((NWIN, WIN), _i32),
            w_b=pltpu.VMEM((NWIN, WIN), _f32),
            idx_b=pltpu.VMEM((NWIN, WIN), _i32),
            val_b=pltpu.VMEM((NWIN, WIN), _f32),
            zb=pltpu.VMEM((CS_PAD // NS,), _f32),
            slotmap=pltpu.VMEM((P,), _i32),
            deg_sh=pltpu.VMEM_SHARED((DEG_PAD,), _f32),
            c_sh=pltpu.VMEM_SHARED((CS_PAD,), _f32),
        ),
        compiler_params=pltpu.CompilerParams(needs_layout_passes=False),
    )
    return kern(src2, dst2, w2, needed)


def _dot_t(a, b):
    return lax.dot_general(a, b, (((1,), (1,)), ((), ())),
                           precision=lax.Precision.HIGHEST,
                           preferred_element_type=_f32)


def _bn_rows(x, g, b):
    mu = jnp.mean(x, axis=0, keepdims=True)
    v = jnp.mean(x * x, axis=0, keepdims=True) - mu * mu
    return (x - mu) * lax.rsqrt(v + EPS) * g + b


def _prep_body(deg_ref, c_ref, pe_ref, needed_ref, ge_ref,
               bn_emb_g, bn_emb_be, sg_w, sg_b,
               fw0, fb0, fg0, fbe0, fw1, fb1, fg1, fbe1,
               bn_pb_g, bn_pb_be,
               a_out, cvec_out):
    deg = deg_ref[0] + deg_ref[1] + 1.0
    dinv = lax.rsqrt(deg + 1e-12)
    c = c_ref[0] + c_ref[1]
    needed = needed_ref[...]
    onehot = (lax.broadcasted_iota(_i32, (P, SLOTS), 0) == needed).astype(_f32)
    dinv_n = jnp.sum(onehot * dinv, axis=0, keepdims=True)
    d = dinv * c * dinv_n + onehot * (dinv_n * dinv_n)
    agg = lax.dot_general(d, pe_ref[...], (((0,), (0,)), ((), ())),
                          precision=lax.Precision.HIGHEST,
                          preferred_element_type=_f32)
    slot_ids = lax.broadcasted_iota(_i32, (SLOTS, SLOTS), 1)
    eq = jnp.transpose(needed) == needed
    rep = jnp.min(jnp.where(eq, slot_ids, SLOTS), axis=1, keepdims=True)
    rmat = (rep == slot_ids).astype(_f32)
    agg_f = jnp.dot(rmat, agg, precision=lax.Precision.HIGHEST,
                    preferred_element_type=_f32)
    asum = jnp.sum(agg_f.reshape(B, 2, H), axis=1)
    pert_sum = _dot_t(asum, sg_w[...]) + 2.0 * sg_b[...]

    t = _dot_t(pert_sum, fw0[...]) + fb0[...]
    t = _bn_rows(t, fg0[...], fbe0[...])
    t = jnp.maximum(t, 0.0)
    t = _dot_t(t, fw1[...]) + fb1[...]
    emb_total = _bn_rows(t, fg1[...], fbe1[...])

    ge = ge_ref[...]
    me = jnp.mean(ge, axis=0, keepdims=True)
    ve = jnp.mean(ge * ge, axis=0, keepdims=True) - me * me
    emb_bn = (ge - me) * lax.rsqrt(ve + EPS) * bn_emb_g[...] + bn_emb_be[...]
    var_embbn = bn_emb_g[...] ** 2 * (ve / (ve + EPS))
    mu_t = jnp.mean(emb_total, axis=0, keepdims=True)
    var_t = jnp.mean(emb_total * emb_total, axis=0, keepdims=True) - mu_t * mu_t
    m_pb = bn_emb_be[...] + mu_t
    t_pb = bn_pb_g[...] * lax.rsqrt(var_embbn + var_t + EPS)
    a_out[...] = emb_bn * t_pb
    cvec_out[...] = emb_total * t_pb + bn_pb_be[...] - m_pb * t_pb


GBS = 1000
NGB = G // GBS
NROWS = float(B * G)


def _z_block(a_ref, cvec_ref):
    z = jnp.maximum(a_ref[...][None, :, :] + cvec_ref[...][:, None, :], 0.0)
    return z.reshape(B * GBS, H)


def _pass1_body(a_ref, cvec_ref, w0, b0, s1_out, s2_out):
    y1 = _dot_t(_z_block(a_ref, cvec_ref), w0[...]) + b0[...]

    @pl.when(pl.program_id(0) == 0)
    def _():
        s1_out[...] = jnp.zeros_like(s1_out)
        s2_out[...] = jnp.zeros_like(s2_out)
    s1_out[...] += jnp.sum(y1, axis=0, keepdims=True)
    s2_out[...] += jnp.sum(y1 * y1, axis=0, keepdims=True)


def _h_block(a_ref, cvec_ref, w0, b0, g0, be0, s1, s2):
    y1 = _dot_t(_z_block(a_ref, cvec_ref), w0[...]) + b0[...]
    m1 = s1[...] / NROWS
    v1 = s2[...] / NROWS - m1 * m1
    t1 = g0[...] * lax.rsqrt(v1 + EPS)
    return jnp.maximum((y1 - m1) * t1 + be0[...], 0.0)


def _pass2_body(a_ref, cvec_ref, w0, b0, g0, be0, s1, s2, w1, b1,
                s1b_out, s2b_out):
    h = _h_block(a_ref, cvec_ref, w0, b0, g0, be0, s1, s2)
    y2 = _dot_t(h, w1[...]) + b1[...]

    @pl.when(pl.program_id(0) == 0)
    def _():
        s1b_out[...] = jnp.zeros_like(s1b_out)
        s2b_out[...] = jnp.zeros_like(s2b_out)
    s1b_out[...] += jnp.sum(y2, axis=0, keepdims=True)
    s2b_out[...] += jnp.sum(y2 * y2, axis=0, keepdims=True)


def _pass3_body(a_ref, cvec_ref, w0, b0, g0, be0, s1, s2, w1, b1,
                s1b, s2b, g1, be1, v1g_ref, b1col_ref, out1_ref):
    h = _h_block(a_ref, cvec_ref, w0, b0, g0, be0, s1, s2)
    y2 = _dot_t(h, w1[...]) + b1[...]
    m2 = s1b[...] / NROWS
    v2 = s2b[...] / NROWS - m2 * m2
    t2 = g1[...] * lax.rsqrt(v2 + EPS)
    vt = v1g_ref[...] * t2
    w = jnp.sum(y2.reshape(B, GBS, H) * vt[None, :, :], axis=2)
    dvec = jnp.sum(v1g_ref[...] * (be1[...] - m2 * t2), axis=1)
    out1_ref[...] = jnp.transpose(w) + dvec[:, None] + b1col_ref[...]


def _final_body(out1t_ref, x2_ref, cw0, cb0, cg0, cbe0, cw1, cb1, cg1, cbe1,
                w2a_ref, w2h_ref, b2_ref, out_ref):
    out1 = jnp.transpose(out1t_ref[...])
    c1 = _dot_t(out1, cw0[...]) + cb0[...]
    c1 = _bn_rows(c1, cg0[...], cbe0[...])
    c1 = jnp.maximum(c1, 0.0)
    c1 = _dot_t(c1, cw1[...]) + cb1[...]
    cg = _bn_rows(c1, cg1[...], cbe1[...])
    out_ref[...] = (out1 * w2a_ref[...] + _dot_t(cg, w2h_ref[...])
                    + b2_ref[...] + x2_ref[...])


def _const_spec(shape):
    return pl.BlockSpec(shape, lambda i: tuple(0 for _ in shape))


def kernel(x, pert_idx, edge_index, edge_weight, params):
    p = params
    src2 = edge_index[0].reshape(NW, NWIN, WIN).astype(_i32)
    dst2 = edge_index[1].reshape(NW, NWIN, WIN).astype(_i32)
    w2 = edge_weight.reshape(NW, NWIN, WIN)
    needed = pert_idx.reshape(2 * B).astype(_i32)

    deg2, c2 = _sc_edges(src2, dst2, w2, needed)
    deg2 = deg2[:, 0, :P].reshape(NC, P, 1)
    c2 = c2[:, 0, :P * SLOTS].reshape(NC, P, SLOTS)

    row = lambda a: a.reshape(1, -1)
    a_mat, cvec = pl.pallas_call(
        _prep_body,
        out_shape=(jax.ShapeDtypeStruct((G, H), _f32),
                   jax.ShapeDtypeStruct((B, H), _f32)),
    )(deg2, c2, p['pert_emb'], needed.reshape(1, 2 * B), p['gene_emb'],
      row(p['bn_emb_g']), row(p['bn_emb_be']), p['sg_W'], row(p['sg_b']),
      p['fuse_W0'], row(p['fuse_b0']), row(p['fuse_g0']), row(p['fuse_be0']),
      p['fuse_W1'], row(p['fuse_b1']), row(p['fuse_g1']), row(p['fuse_be1']),
      row(p['bn_pb_g']), row(p['bn_pb_be']))

    a_spec = pl.BlockSpec((GBS, H), lambda i: (i, 0))
    grid = (NGB,)
    w0, b0 = p['rec_W0'], row(p['rec_b0'])
    g0, be0 = row(p['rec_g0']), row(p['rec_be0'])
    w1, b1 = p['rec_W1'], row(p['rec_b1'])
    g1, be1 = row(p['rec_g1']), row(p['rec_be1'])

    s1, s2 = pl.pallas_call(
        _pass1_body,
        grid=grid,
        in_specs=[a_spec, _const_spec((B, H)), _const_spec((2 * H, H)),
                  _const_spec((1, 2 * H))],
        out_specs=(_const_spec((1, 2 * H)), _const_spec((1, 2 * H))),
        out_shape=(jax.ShapeDtypeStruct((1, 2 * H), _f32),
                   jax.ShapeDtypeStruct((1, 2 * H), _f32)),
    )(a_mat, cvec, w0, b0)

    s1b, s2b = pl.pallas_call(
        _pass2_body,
        grid=grid,
        in_specs=[a_spec, _const_spec((B, H)), _const_spec((2 * H, H)),
                  _const_spec((1, 2 * H)), _const_spec((1, 2 * H)),
                  _const_spec((1, 2 * H)), _const_spec((1, 2 * H)),
                  _const_spec((1, 2 * H)), _const_spec((H, 2 * H)),
                  _const_spec((1, H))],
        out_specs=(_const_spec((1, H)), _const_spec((1, H))),
        out_shape=(jax.ShapeDtypeStruct((1, H), _f32),
                   jax.ShapeDtypeStruct((1, H), _f32)),
    )(a_mat, cvec, w0, b0, g0, be0, s1, s2, w1, b1)

    v1g = p['indv_w1'][:, :, 0]
    b1col = p['indv_b1']
    out1t = pl.pallas_call(
        _pass3_body,
        grid=grid,
        in_specs=[a_spec, _const_spec((B, H)), _const_spec((2 * H, H)),
                  _const_spec((1, 2 * H)), _const_spec((1, 2 * H)),
                  _const_spec((1, 2 * H)), _const_spec((1, 2 * H)),
                  _const_spec((1, 2 * H)), _const_spec((H, 2 * H)),
                  _const_spec((1, H)), _const_spec((1, H)), _const_spec((1, H)),
                  _const_spec((1, H)), _const_spec((1, H)),
                  pl.BlockSpec((GBS, H), lambda i: (i, 0)),
                  pl.BlockSpec((GBS, 1), lambda i: (i, 0))],
        out_specs=pl.BlockSpec((GBS, B), lambda i: (i, 0)),
        out_shape=jax.ShapeDtypeStruct((G, B), _f32),
    )(a_mat, cvec, w0, b0, g0, be0, s1, s2, w1, b1,
      s1b, s2b, g1, be1, v1g, b1col)

    x2 = x.reshape(B, G + 1)[:, :-1]
    w2a = p['indv_w2'][0, :, 0].reshape(1, G)
    w2h = p['indv_w2'][0, :, 1:]
    b2row = p['indv_b2'][0].reshape(1, G)
    final = pl.pallas_call(
        _final_body,
        out_shape=jax.ShapeDtypeStruct((B, G), _f32),
    )(out1t, x2, p['cg_W0'], row(p['cg_b0']), row(p['cg_g0']), row(p['cg_be0']),
      p['cg_W1'], row(p['cg_b1']), row(p['cg_g1']), row(p['cg_be1']),
      w2a, w2h, b2row)
    return final


# speedup vs baseline: 2.3949x; 2.3949x over previous
import functools

import jax
import jax.numpy as jnp
from jax import lax
from jax.experimental import pallas as pl
from jax.experimental.pallas import tpu as pltpu
import jax.experimental.pallas.tpu_sc as plsc

G = 5000
P = 10000
H = 128
B = 8
E = 320000
EPS = 1e-5

NC, NS = 2, 16
NW = NC * NS
EPW = E // NW
WIN = 80
NWIN = EPW // WIN
DEG_PAD = 10240
SLOTS = 16
CS_PAD = 16 * 10240

_f32 = jnp.float32
_i32 = jnp.int32


def _sc_body(src_h, dst_h, w_h, needed_h, deg_out, c_out,
             needed_v, src_b, dst_b, w_b, idx_b, val_b, zb, slotmap,
             deg_sh, c_sh):
    cid = lax.axis_index("c")
    sid = lax.axis_index("s")
    wid = cid * NS + sid

    def _z(i, _):
        zb[pl.ds(i * 16, 16)] = jnp.zeros((16,), _f32)
        return 0
    lax.fori_loop(0, (CS_PAD // NS) // 16, _z, 0)
    pltpu.sync_copy(zb, c_sh.at[pl.ds(sid * (CS_PAD // NS), CS_PAD // NS)])
    pltpu.sync_copy(zb.at[pl.ds(0, DEG_PAD // NS)],
                    deg_sh.at[pl.ds(sid * (DEG_PAD // NS), DEG_PAD // NS)])

    pltpu.sync_copy(needed_h, needed_v)
    pltpu.sync_copy(src_h.at[wid], src_b)
    pltpu.sync_copy(dst_h.at[wid], dst_b)
    pltpu.sync_copy(w_h.at[wid], w_b)

    def _zs(i, _):
        slotmap[pl.ds(i * 16, 16)] = jnp.zeros((16,), _i32)
        return 0
    lax.fori_loop(0, P // 16, _zs, 0)
    needed_vec = needed_v[...]
    repv = jnp.full((16,), SLOTS, _i32)
    for s in range(SLOTS):
        ns = needed_vec[s]
        repv = jnp.minimum(repv, jnp.where(needed_vec == ns, s, SLOTS))
    plsc.store_scatter(slotmap, [needed_vec], repv + 1)

    plsc.subcore_barrier()

    def _compute(j, _):
        for k in range(WIN // 16):
            off = k * 16
            srcv = src_b[j, pl.ds(off, 16)]
            dstv = dst_b[j, pl.ds(off, 16)]
            wv = w_b[j, pl.ds(off, 16)]
            slotv = plsc.load_gather(slotmap, [dstv])
            idx = srcv * SLOTS + jnp.maximum(slotv - 1, 0)
            val = jnp.where(slotv > 0, wv, jnp.zeros((16,), _f32))
            idx_b[j, pl.ds(off, 16)] = idx
            val_b[j, pl.ds(off, 16)] = val
        return 0
    lax.fori_loop(0, NWIN, _compute, 0)

    def _scatter(j, _):
        pltpu.sync_copy(w_b.at[j], deg_sh.at[dst_b.at[j]], add=True)
        pltpu.sync_copy(val_b.at[j], c_sh.at[idx_b.at[j]], add=True)
        return 0
    lax.fori_loop(0, NWIN, _scatter, 0)

    plsc.subcore_barrier()

    pltpu.sync_copy(deg_sh.at[pl.ds(sid * (DEG_PAD // NS), DEG_PAD // NS)],
                    deg_out.at[cid, 0, pl.ds(sid * (DEG_PAD // NS), DEG_PAD // NS)])
    pltpu.sync_copy(c_sh.at[pl.ds(sid * (CS_PAD // NS), CS_PAD // NS)],
                    c_out.at[cid, 0, pl.ds(sid * (CS_PAD // NS), CS_PAD // NS)])


def _sc_edges(src2, dst2, w2, needed):
    mesh = plsc.VectorSubcoreMesh(core_axis_name="c", subcore_axis_name="s",
                                  num_cores=NC, num_subcores=NS)
    kern = pl.kernel(
        _sc_body,
        out_type=(jax.ShapeDtypeStruct((NC, 1, DEG_PAD), _f32),
                  jax.ShapeDtypeStruct((NC, 1, CS_PAD), _f32)),
        mesh=mesh,
        scratch_types=dict(
            needed_v=pltpu.VMEM((16,), _i32),
            src_b=pltpu.VMEM((NWIN, WIN), _i32),
            dst_b=pltpu.VMEM((NWIN, WIN), _i32),
            w_b=pltpu.VMEM((NWIN, WIN), _f32),
            idx_b=pltpu.VMEM((NWIN, WIN), _i32),
            val_b=pltpu.VMEM((NWIN, WIN), _f32),
            zb=pltpu.VMEM((CS_PAD // NS,), _f32),
            slotmap=pltpu.VMEM((P,), _i32),
            deg_sh=pltpu.VMEM_SHARED((DEG_PAD,), _f32),
            c_sh=pltpu.VMEM_SHARED((CS_PAD,), _f32),
        ),
        compiler_params=pltpu.CompilerParams(needs_layout_passes=False),
    )
    return kern(src2, dst2, w2, needed)


def _dot_t(a, b, precision=None):
    return lax.dot_general(a, b, (((1,), (1,)), ((), ())),
                           precision=precision,
                           preferred_element_type=_f32)


def _dot_th(a, b):
    return _dot_t(a, b, precision=lax.Precision.HIGHEST)


def _bn_rows(x, g, b):
    mu = jnp.mean(x, axis=0, keepdims=True)
    d = x - mu
    v = jnp.mean(d * d, axis=0, keepdims=True)
    return d * lax.rsqrt(v + EPS) * g + b


def _prep_body(deg_ref, c_ref, pe_ref, needed_ref, ge_ref,
               bn_emb_g, bn_emb_be, sg_w, sg_b,
               fw0, fb0, fg0, fbe0, fw1, fb1, fg1, fbe1,
               bn_pb_g, bn_pb_be,
               a_out, cvec_out):
    deg = deg_ref[0] + deg_ref[1] + 1.0
    dinv = lax.rsqrt(deg + 1e-12)
    c = c_ref[0] + c_ref[1]
    needed = needed_ref[...]
    onehot = (lax.broadcasted_iota(_i32, (P, SLOTS), 0) == needed).astype(_f32)
    dinv_n = jnp.sum(onehot * dinv, axis=0, keepdims=True)
    d = dinv * c * dinv_n + onehot * (dinv_n * dinv_n)
    agg = lax.dot_general(d, pe_ref[...], (((0,), (0,)), ((), ())),
                          precision=lax.Precision.HIGHEST,
                          preferred_element_type=_f32)
    slot_ids = lax.broadcasted_iota(_i32, (SLOTS, SLOTS), 1)
    eq = jnp.transpose(needed) == needed
    rep = jnp.min(jnp.where(eq, slot_ids, SLOTS), axis=1, keepdims=True)
    rmat = (rep == slot_ids).astype(_f32)
    agg_f = jnp.dot(rmat, agg, precision=lax.Precision.HIGHEST,
                    preferred_element_type=_f32)
    asum = jnp.sum(agg_f.reshape(B, 2, H), axis=1)
    pert_sum = _dot_th(asum, sg_w[...]) + 2.0 * sg_b[...]

    t = _dot_th(pert_sum, fw0[...]) + fb0[...]
    t = _bn_rows(t, fg0[...], fbe0[...])
    t = jnp.maximum(t, 0.0)
    t = _dot_th(t, fw1[...]) + fb1[...]
    emb_total = _bn_rows(t, fg1[...], fbe1[...])

    ge = ge_ref[...]
    me = jnp.mean(ge, axis=0, keepdims=True)
    ve = jnp.mean(ge * ge, axis=0, keepdims=True) - me * me
    emb_bn = (ge - me) * lax.rsqrt(ve + EPS) * bn_emb_g[...] + bn_emb_be[...]
    var_embbn = bn_emb_g[...] ** 2 * (ve / (ve + EPS))
    mu_t = jnp.mean(emb_total, axis=0, keepdims=True)
    var_t = jnp.mean(emb_total * emb_total, axis=0, keepdims=True) - mu_t * mu_t
    m_pb = bn_emb_be[...] + mu_t
    t_pb = bn_pb_g[...] * lax.rsqrt(var_embbn + var_t + EPS)
    a_out[...] = emb_bn * t_pb
    cvec_out[...] = emb_total * t_pb + bn_pb_be[...] - m_pb * t_pb


GBS = 1000
NGB = G // GBS
NROWS = float(B * G)


def _z_block(a_ref, cvec_ref):
    z = jnp.maximum(a_ref[...][None, :, :] + cvec_ref[...][:, None, :], 0.0)
    return z.reshape(B * GBS, H)


def _pass1_body(a_ref, cvec_ref, w0, b0, s1_out, s2_out):
    y1 = _dot_t(_z_block(a_ref, cvec_ref), w0[...]) + b0[...]

    @pl.when(pl.program_id(0) == 0)
    def _():
        s1_out[...] = jnp.zeros_like(s1_out)
        s2_out[...] = jnp.zeros_like(s2_out)
    s1_out[...] += jnp.sum(y1, axis=0, keepdims=True)
    s2_out[...] += jnp.sum(y1 * y1, axis=0, keepdims=True)


def _h_block(a_ref, cvec_ref, w0, b0, g0, be0, s1, s2):
    y1 = _dot_t(_z_block(a_ref, cvec_ref), w0[...]) + b0[...]
    m1 = s1[...] / NROWS
    v1 = s2[...] / NROWS - m1 * m1
    t1 = g0[...] * lax.rsqrt(v1 + EPS)
    return jnp.maximum((y1 - m1) * t1 + be0[...], 0.0)


def _pass2_body(a_ref, cvec_ref, w0, b0, g0, be0, s1, s2, w1, b1,
                s1b_out, s2b_out):
    h = _h_block(a_ref, cvec_ref, w0, b0, g0, be0, s1, s2)
    y2 = _dot_t(h, w1[...]) + b1[...]

    @pl.when(pl.program_id(0) == 0)
    def _():
        s1b_out[...] = jnp.zeros_like(s1b_out)
        s2b_out[...] = jnp.zeros_like(s2b_out)
    s1b_out[...] += jnp.sum(y2, axis=0, keepdims=True)
    s2b_out[...] += jnp.sum(y2 * y2, axis=0, keepdims=True)


def _pass3_body(a_ref, cvec_ref, w0, b0, g0, be0, s1, s2, w1, b1,
                s1b, s2b, g1, be1, v1g_ref, b1col_ref, out1_ref):
    h = _h_block(a_ref, cvec_ref, w0, b0, g0, be0, s1, s2)
    y2 = _dot_t(h, w1[...]) + b1[...]
    m2 = s1b[...] / NROWS
    v2 = s2b[...] / NROWS - m2 * m2
    t2 = g1[...] * lax.rsqrt(v2 + EPS)
    vt = v1g_ref[...] * t2
    w = jnp.sum(y2.reshape(B, GBS, H) * vt[None, :, :], axis=2)
    dvec = jnp.sum(v1g_ref[...] * (be1[...] - m2 * t2), axis=1)
    out1_ref[...] = jnp.transpose(w) + dvec[:, None] + b1col_ref[...]


def _final_body(out1t_ref, x2_ref, cw0, cb0, cg0, cbe0, cw1, cb1, cg1, cbe1,
                w2a_ref, w2h_ref, b2_ref, out_ref):
    out1 = jnp.transpose(out1t_ref[...])
    c1 = _dot_th(out1, cw0[...]) + cb0[...]
    c1 = _bn_rows(c1, cg0[...], cbe0[...])
    c1 = jnp.maximum(c1, 0.0)
    c1 = _dot_th(c1, cw1[...]) + cb1[...]
    cg = _bn_rows(c1, cg1[...], cbe1[...])
    out_ref[...] = (out1 * w2a_ref[...] + _dot_th(cg, w2h_ref[...])
                    + b2_ref[...] + x2_ref[...])


def _const_spec(shape):
    return pl.BlockSpec(shape, lambda i: tuple(0 for _ in shape))


def kernel(x, pert_idx, edge_index, edge_weight, params):
    p = params
    src2 = edge_index[0].reshape(NW, NWIN, WIN).astype(_i32)
    dst2 = edge_index[1].reshape(NW, NWIN, WIN).astype(_i32)
    w2 = edge_weight.reshape(NW, NWIN, WIN)
    needed = pert_idx.reshape(2 * B).astype(_i32)

    deg2, c2 = _sc_edges(src2, dst2, w2, needed)
    deg2 = deg2[:, 0, :P].reshape(NC, P, 1)
    c2 = c2[:, 0, :P * SLOTS].reshape(NC, P, SLOTS)

    row = lambda a: a.reshape(1, -1)
    a_mat, cvec = pl.pallas_call(
        _prep_body,
        out_shape=(jax.ShapeDtypeStruct((G, H), _f32),
                   jax.ShapeDtypeStruct((B, H), _f32)),
    )(deg2, c2, p['pert_emb'], needed.reshape(1, 2 * B), p['gene_emb'],
      row(p['bn_emb_g']), row(p['bn_emb_be']), p['sg_W'], row(p['sg_b']),
      p['fuse_W0'], row(p['fuse_b0']), row(p['fuse_g0']), row(p['fuse_be0']),
      p['fuse_W1'], row(p['fuse_b1']), row(p['fuse_g1']), row(p['fuse_be1']),
      row(p['bn_pb_g']), row(p['bn_pb_be']))

    a_spec = pl.BlockSpec((GBS, H), lambda i: (i, 0))
    grid = (NGB,)
    w0, b0 = p['rec_W0'], row(p['rec_b0'])
    g0, be0 = row(p['rec_g0']), row(p['rec_be0'])
    w1, b1 = p['rec_W1'], row(p['rec_b1'])
    g1, be1 = row(p['rec_g1']), row(p['rec_be1'])

    s1, s2 = pl.pallas_call(
        _pass1_body,
        grid=grid,
        in_specs=[a_spec, _const_spec((B, H)), _const_spec((2 * H, H)),
                  _const_spec((1, 2 * H))],
        out_specs=(_const_spec((1, 2 * H)), _const_spec((1, 2 * H))),
        out_shape=(jax.ShapeDtypeStruct((1, 2 * H), _f32),
                   jax.ShapeDtypeStruct((1, 2 * H), _f32)),
    )(a_mat, cvec, w0, b0)

    s1b, s2b = pl.pallas_call(
        _pass2_body,
        grid=grid,
        in_specs=[a_spec, _const_spec((B, H)), _const_spec((2 * H, H)),
                  _const_spec((1, 2 * H)), _const_spec((1, 2 * H)),
                  _const_spec((1, 2 * H)), _const_spec((1, 2 * H)),
                  _const_spec((1, 2 * H)), _const_spec((H, 2 * H)),
                  _const_spec((1, H))],
        out_specs=(_const_spec((1, H)), _const_spec((1, H))),
        out_shape=(jax.ShapeDtypeStruct((1, H), _f32),
                   jax.ShapeDtypeStruct((1, H), _f32)),
    )(a_mat, cvec, w0, b0, g0, be0, s1, s2, w1, b1)

    v1g = p['indv_w1'][:, :, 0]
    b1col = p['indv_b1']
    out1t = pl.pallas_call(
        _pass3_body,
        grid=grid,
        in_specs=[a_spec, _const_spec((B, H)), _const_spec((2 * H, H)),
                  _const_spec((1, 2 * H)), _const_spec((1, 2 * H)),
                  _const_spec((1, 2 * H)), _const_spec((1, 2 * H)),
                  _const_spec((1, 2 * H)), _const_spec((H, 2 * H)),
                  _const_spec((1, H)), _const_spec((1, H)), _const_spec((1, H)),
                  _const_spec((1, H)), _const_spec((1, H)),
                  pl.BlockSpec((GBS, H), lambda i: (i, 0)),
                  pl.BlockSpec((GBS, 1), lambda i: (i, 0))],
        out_specs=pl.BlockSpec((GBS, B), lambda i: (i, 0)),
        out_shape=jax.ShapeDtypeStruct((G, B), _f32),
    )(a_mat, cvec, w0, b0, g0, be0, s1, s2, w1, b1,
      s1b, s2b, g1, be1, v1g, b1col)

    x2 = x.reshape(B, G + 1)[:, :-1]
    w2a = p['indv_w2'][0, :, 0].reshape(1, G)
    w2h = p['indv_w2'][0, :, 1:]
    b2row = p['indv_b2'][0].reshape(1, G)
    final = pl.pallas_call(
        _final_body,
        out_shape=jax.ShapeDtypeStruct((B, G), _f32),
    )(out1t, x2, p['cg_W0'], row(p['cg_b0']), row(p['cg_g0']), row(p['cg_be0']),
      p['cg_W1'], row(p['cg_b1']), row(p['cg_g1']), row(p['cg_be1']),
      w2a, w2h, b2row)
    return final


# speedup vs baseline: 2.5071x; 1.0468x over previous
import functools

import jax
import jax.numpy as jnp
from jax import lax
from jax.experimental import pallas as pl
from jax.experimental.pallas import tpu as pltpu
import jax.experimental.pallas.tpu_sc as plsc

G = 5000
P = 10000
H = 128
B = 8
E = 320000
EPS = 1e-5

NC, NS = 2, 16
NW = NC * NS
EPW = E // NW
WIN = 80
NWIN = EPW // WIN
DEG_PAD = 10240
SLOTS = 16
CS_PAD = 16 * 10240

_f32 = jnp.float32
_i32 = jnp.int32


def _sc_body(src_h, dst_h, w_h, needed_h, deg_out, c_out,
             needed_v, src_b, dst_b, w_b, idx_b, val_b, zb, slotmap,
             deg_sh, c_sh):
    cid = lax.axis_index("c")
    sid = lax.axis_index("s")
    wid = cid * NS + sid

    def _z(i, _):
        zb[pl.ds(i * 16, 16)] = jnp.zeros((16,), _f32)
        return 0
    lax.fori_loop(0, (CS_PAD // NS) // 16, _z, 0)
    pltpu.sync_copy(zb, c_sh.at[pl.ds(sid * (CS_PAD // NS), CS_PAD // NS)])
    pltpu.sync_copy(zb.at[pl.ds(0, DEG_PAD // NS)],
                    deg_sh.at[pl.ds(sid * (DEG_PAD // NS), DEG_PAD // NS)])

    pltpu.sync_copy(needed_h, needed_v)
    pltpu.sync_copy(src_h.at[wid], src_b)
    pltpu.sync_copy(dst_h.at[wid], dst_b)
    pltpu.sync_copy(w_h.at[wid], w_b)

    def _zs(i, _):
        slotmap[pl.ds(i * 16, 16)] = jnp.zeros((16,), _i32)
        return 0
    lax.fori_loop(0, P // 16, _zs, 0)
    needed_vec = needed_v[...]
    repv = jnp.full((16,), SLOTS, _i32)
    for s in range(SLOTS):
        ns = needed_vec[s]
        repv = jnp.minimum(repv, jnp.where(needed_vec == ns, s, SLOTS))
    plsc.store_scatter(slotmap, [needed_vec], repv + 1)

    plsc.subcore_barrier()

    def _compute(j, _):
        for k in range(WIN // 16):
            off = k * 16
            srcv = src_b[j, pl.ds(off, 16)]
            dstv = dst_b[j, pl.ds(off, 16)]
            wv = w_b[j, pl.ds(off, 16)]
            slotv = plsc.load_gather(slotmap, [dstv])
            idx = srcv * SLOTS + jnp.maximum(slotv - 1, 0)
            val = jnp.where(slotv > 0, wv, jnp.zeros((16,), _f32))
            idx_b[j, pl.ds(off, 16)] = idx
            val_b[j, pl.ds(off, 16)] = val
        return 0
    lax.fori_loop(0, NWIN, _compute, 0)

    def _scatter(j, _):
        pltpu.sync_copy(w_b.at[j], deg_sh.at[dst_b.at[j]], add=True)
        pltpu.sync_copy(val_b.at[j], c_sh.at[idx_b.at[j]], add=True)
        return 0
    lax.fori_loop(0, NWIN, _scatter, 0)

    plsc.subcore_barrier()

    pltpu.sync_copy(deg_sh.at[pl.ds(sid * (DEG_PAD // NS), DEG_PAD // NS)],
                    deg_out.at[cid, 0, pl.ds(sid * (DEG_PAD // NS), DEG_PAD // NS)])
    pltpu.sync_copy(c_sh.at[pl.ds(sid * (CS_PAD // NS), CS_PAD // NS)],
                    c_out.at[cid, 0, pl.ds(sid * (CS_PAD // NS), CS_PAD // NS)])


def _sc_edges(src2, dst2, w2, needed):
    mesh = plsc.VectorSubcoreMesh(core_axis_name="c", subcore_axis_name="s",
                                  num_cores=NC, num_subcores=NS)
    kern = pl.kernel(
        _sc_body,
        out_type=(jax.ShapeDtypeStruct((NC, 1, DEG_PAD), _f32),
                  jax.ShapeDtypeStruct((NC, 1, CS_PAD), _f32)),
        mesh=mesh,
        scratch_types=dict(
            needed_v=pltpu.VMEM((16,), _i32),
            src_b=pltpu.VMEM((NWIN, WIN), _i32),
            dst_b=pltpu.VMEM((NWIN, WIN), _i32),
            w_b=pltpu.VMEM((NWIN, WIN), _f32),
            idx_b=pltpu.VMEM((NWIN, WIN), _i32),
            val_b=pltpu.VMEM((NWIN, WIN), _f32),
            zb=pltpu.VMEM((CS_PAD // NS,), _f32),
            slotmap=pltpu.VMEM((P,), _i32),
            deg_sh=pltpu.VMEM_SHARED((DEG_PAD,), _f32),
            c_sh=pltpu.VMEM_SHARED((CS_PAD,), _f32),
        ),
        compiler_params=pltpu.CompilerParams(needs_layout_passes=False),
    )
    return kern(src2, dst2, w2, needed)


def _dot_t(a, b, precision=None):
    return lax.dot_general(a, b, (((1,), (1,)), ((), ())),
                           precision=precision,
                           preferred_element_type=_f32)


def _dot_th(a, b):
    return _dot_t(a, b, precision=lax.Precision.HIGHEST)


def _bn_rows(x, g, b):
    mu = jnp.mean(x, axis=0, keepdims=True)
    d = x - mu
    v = jnp.mean(d * d, axis=0, keepdims=True)
    return d * lax.rsqrt(v + EPS) * g + b


GBS = 1000
NGB = G // GBS
NROWS = float(B * G)


def _stage1_body(ge_blk, ge_full, deg_ref, c_ref, pe_ref, needed_ref,
                 bn_emb_g, bn_emb_be, sg_w, sg_b,
                 fw0, fb0, fg0, fbe0, fw1, fb1, fg1, fbe1,
                 bn_pb_g, bn_pb_be, w0, b0,
                 alpha_out, cvec2_out, s1_out, s2_out):
    i = pl.program_id(0)

    @pl.when(i == 0)
    def _():
        deg = deg_ref[0] + deg_ref[1] + 1.0
        dinv = lax.rsqrt(deg + 1e-12)
        c = c_ref[0] + c_ref[1]
        needed = needed_ref[...]
        onehot = (lax.broadcasted_iota(_i32, (P, SLOTS), 0) == needed).astype(_f32)
        dinv_n = jnp.sum(onehot * dinv, axis=0, keepdims=True)
        d = dinv * c * dinv_n + onehot * (dinv_n * dinv_n)
        agg = lax.dot_general(d, pe_ref[...], (((0,), (0,)), ((), ())),
                              precision=lax.Precision.HIGHEST,
                              preferred_element_type=_f32)
        slot_ids = lax.broadcasted_iota(_i32, (SLOTS, SLOTS), 1)
        eq = jnp.transpose(needed) == needed
        rep = jnp.min(jnp.where(eq, slot_ids, SLOTS), axis=1, keepdims=True)
        rmat = (rep == slot_ids).astype(_f32)
        agg_f = jnp.dot(rmat, agg, precision=lax.Precision.HIGHEST,
                        preferred_element_type=_f32)
        asum = jnp.sum(agg_f.reshape(B, 2, H), axis=1)
        pert_sum = _dot_th(asum, sg_w[...]) + 2.0 * sg_b[...]

        t = _dot_th(pert_sum, fw0[...]) + fb0[...]
        t = _bn_rows(t, fg0[...], fbe0[...])
        t = jnp.maximum(t, 0.0)
        t = _dot_th(t, fw1[...]) + fb1[...]
        emb_total = _bn_rows(t, fg1[...], fbe1[...])

        ge = ge_full[...]
        me = jnp.mean(ge, axis=0, keepdims=True)
        ve = jnp.mean(ge * ge, axis=0, keepdims=True) - me * me
        rs = lax.rsqrt(ve + EPS)
        var_embbn = bn_emb_g[...] ** 2 * (ve / (ve + EPS))
        mu_t = jnp.mean(emb_total, axis=0, keepdims=True)
        dt = emb_total - mu_t
        var_t = jnp.mean(dt * dt, axis=0, keepdims=True)
        m_pb = bn_emb_be[...] + mu_t
        t_pb = bn_pb_g[...] * lax.rsqrt(var_embbn + var_t + EPS)
        alpha = rs * bn_emb_g[...] * t_pb
        beta = (bn_emb_be[...] - me * rs * bn_emb_g[...]) * t_pb
        cvec = emb_total * t_pb + bn_pb_be[...] - m_pb * t_pb
        alpha_out[...] = alpha
        cvec2_out[...] = cvec + beta
        s1_out[...] = jnp.zeros_like(s1_out)
        s2_out[...] = jnp.zeros_like(s2_out)

    z = jnp.maximum(ge_blk[...][None, :, :] * alpha_out[...]
                    + cvec2_out[...][:, None, :], 0.0).reshape(B * GBS, H)
    y1 = _dot_t(z, w0[...]) + b0[...]
    s1_out[...] += jnp.sum(y1, axis=0, keepdims=True)
    s2_out[...] += jnp.sum(y1 * y1, axis=0, keepdims=True)


def _stage2_body(ge_blk, alpha, cvec2, w0, b0, g0, be0, s1, s2, w1, b1,
                 y2_out, s1b_out, s2b_out):
    z = jnp.maximum(ge_blk[...][None, :, :] * alpha[...]
                    + cvec2[...][:, None, :], 0.0).reshape(B * GBS, H)
    y1 = _dot_t(z, w0[...]) + b0[...]
    m1 = s1[...] / NROWS
    v1 = s2[...] / NROWS - m1 * m1
    t1 = g0[...] * lax.rsqrt(v1 + EPS)
    h = jnp.maximum((y1 - m1) * t1 + be0[...], 0.0)
    y2 = _dot_t(h, w1[...]) + b1[...]
    y2_out[...] = y2.reshape(B, GBS, H)

    @pl.when(pl.program_id(0) == 0)
    def _():
        s1b_out[...] = jnp.zeros_like(s1b_out)
        s2b_out[...] = jnp.zeros_like(s2b_out)
    s1b_out[...] += jnp.sum(y2, axis=0, keepdims=True)
    s2b_out[...] += jnp.sum(y2 * y2, axis=0, keepdims=True)


def _reduce_body(y2_ref, s1b, s2b, g1, be1, v1g_ref, b1col_ref, out1t_ref):
    m2 = s1b[...] / NROWS
    v2 = s2b[...] / NROWS - m2 * m2
    t2 = g1[...] * lax.rsqrt(v2 + EPS)
    v1g = v1g_ref[...]
    vt = v1g * t2
    w = jnp.sum(y2_ref[...] * vt[None, :, :], axis=2)
    dvec = jnp.sum(v1g * (be1[...] - m2 * t2), axis=1, keepdims=True)
    out1t_ref[...] = jnp.transpose(w) + dvec + b1col_ref[...]


def _final_body(out1t_ref, x2_ref, cw0, cb0, cg0, cbe0, cw1, cb1, cg1, cbe1,
                w2a_ref, w2h_ref, b2_ref, out_ref):
    out1 = jnp.transpose(out1t_ref[...])
    c1 = _dot_th(out1, cw0[...]) + cb0[...]
    c1 = _bn_rows(c1, cg0[...], cbe0[...])
    c1 = jnp.maximum(c1, 0.0)
    c1 = _dot_th(c1, cw1[...]) + cb1[...]
    cg = _bn_rows(c1, cg1[...], cbe1[...])
    out_ref[...] = (out1 * w2a_ref[...] + _dot_th(cg, w2h_ref[...])
                    + b2_ref[...] + x2_ref[...])


def _const_spec(shape):
    return pl.BlockSpec(shape, lambda i: tuple(0 for _ in shape))


def kernel(x, pert_idx, edge_index, edge_weight, params):
    p = params
    src2 = edge_index[0].reshape(NW, NWIN, WIN).astype(_i32)
    dst2 = edge_index[1].reshape(NW, NWIN, WIN).astype(_i32)
    w2 = edge_weight.reshape(NW, NWIN, WIN)
    needed = pert_idx.reshape(2 * B).astype(_i32)

    deg2, c2 = _sc_edges(src2, dst2, w2, needed)
    deg2 = deg2[:, 0, :P].reshape(NC, P, 1)
    c2 = c2[:, 0, :P * SLOTS].reshape(NC, P, SLOTS)

    row = lambda a: a.reshape(1, -1)
    ge_spec = pl.BlockSpec((GBS, H), lambda i: (i, 0))
    grid = (NGB,)
    w0, b0 = p['rec_W0'], row(p['rec_b0'])
    g0, be0 = row(p['rec_g0']), row(p['rec_be0'])
    w1, b1 = p['rec_W1'], row(p['rec_b1'])
    g1, be1 = row(p['rec_g1']), row(p['rec_be1'])

    alpha, cvec2, s1, s2 = pl.pallas_call(
        _stage1_body,
        grid=grid,
        in_specs=[ge_spec, _const_spec((G, H)), _const_spec((NC, P, 1)),
                  _const_spec((NC, P, SLOTS)), _const_spec((P, H)),
                  _const_spec((1, 2 * B)),
                  _const_spec((1, H)), _const_spec((1, H)),
                  _const_spec((H, H)), _const_spec((1, H)),
                  _const_spec((H, H)), _const_spec((1, H)),
                  _const_spec((1, H)), _const_spec((1, H)),
                  _const_spec((H, H)), _const_spec((1, H)),
                  _const_spec((1, H)), _const_spec((1, H)),
                  _const_spec((1, H)), _const_spec((1, H)),
                  _const_spec((2 * H, H)), _const_spec((1, 2 * H))],
        out_specs=(_const_spec((1, H)), _const_spec((B, H)),
                   _const_spec((1, 2 * H)), _const_spec((1, 2 * H))),
        out_shape=(jax.ShapeDtypeStruct((1, H), _f32),
                   jax.ShapeDtypeStruct((B, H), _f32),
                   jax.ShapeDtypeStruct((1, 2 * H), _f32),
                   jax.ShapeDtypeStruct((1, 2 * H), _f32)),
    )(p['gene_emb'], p['gene_emb'], deg2, c2, p['pert_emb'],
      needed.reshape(1, 2 * B),
      row(p['bn_emb_g']), row(p['bn_emb_be']), p['sg_W'], row(p['sg_b']),
      p['fuse_W0'], row(p['fuse_b0']), row(p['fuse_g0']), row(p['fuse_be0']),
      p['fuse_W1'], row(p['fuse_b1']), row(p['fuse_g1']), row(p['fuse_be1']),
      row(p['bn_pb_g']), row(p['bn_pb_be']), w0, b0)

    y2, s1b, s2b = pl.pallas_call(
        _stage2_body,
        grid=grid,
        in_specs=[ge_spec, _const_spec((1, H)), _const_spec((B, H)),
                  _const_spec((2 * H, H)), _const_spec((1, 2 * H)),
                  _const_spec((1, 2 * H)), _const_spec((1, 2 * H)),
                  _const_spec((1, 2 * H)), _const_spec((1, 2 * H)),
                  _const_spec((H, 2 * H)), _const_spec((1, H))],
        out_specs=(pl.BlockSpec((B, GBS, H), lambda i: (0, i, 0)),
                   _const_spec((1, H)), _const_spec((1, H))),
        out_shape=(jax.ShapeDtypeStruct((B, G, H), _f32),
                   jax.ShapeDtypeStruct((1, H), _f32),
                   jax.ShapeDtypeStruct((1, H), _f32)),
    )(p['gene_emb'], alpha, cvec2, w0, b0, g0, be0, s1, s2, w1, b1)

    v1g = p['indv_w1'][:, :, 0]
    b1col = p['indv_b1']
    out1t = pl.pallas_call(
        _reduce_body,
        grid=grid,
        in_specs=[pl.BlockSpec((B, GBS, H), lambda i: (0, i, 0)),
                  _const_spec((1, H)), _const_spec((1, H)),
                  _const_spec((1, H)), _const_spec((1, H)),
                  pl.BlockSpec((GBS, H), lambda i: (i, 0)),
                  pl.BlockSpec((GBS, 1), lambda i: (i, 0))],
        out_specs=pl.BlockSpec((GBS, B), lambda i: (i, 0)),
        out_shape=jax.ShapeDtypeStruct((G, B), _f32),
    )(y2, s1b, s2b, g1, be1, v1g, b1col)

    x2 = x.reshape(B, G + 1)[:, :-1]
    w2a = p['indv_w2'][0, :, 0].reshape(1, G)
    w2h = p['indv_w2'][0, :, 1:]
    b2row = p['indv_b2'][0].reshape(1, G)
    final = pl.pallas_call(
        _final_body,
        out_shape=jax.ShapeDtypeStruct((B, G), _f32),
    )(out1t, x2,
      p['cg_W0'], row(p['cg_b0']), row(p['cg_g0']), row(p['cg_be0']),
      p['cg_W1'], row(p['cg_b1']), row(p['cg_g1']), row(p['cg_be1']),
      w2a, w2h, b2row)
    return final


# speedup vs baseline: 2.7296x; 1.0888x over previous
import functools

import jax
import jax.numpy as jnp
from jax import lax
from jax.experimental import pallas as pl
from jax.experimental.pallas import tpu as pltpu
import jax.experimental.pallas.tpu_sc as plsc

G = 5000
P = 10000
H = 128
B = 8
E = 320000
EPS = 1e-5

NC, NS = 2, 16
NW = NC * NS
EPW = E // NW
WIN = 80
NWIN = EPW // WIN
WPC = 5
NCHUNK = NWIN // WPC
DEG_PAD = 10240
SLOTS = 16
CS_PAD = 16 * 10240

_f32 = jnp.float32
_i32 = jnp.int32


def _sc_body(src_h, dst_h, w_h, needed_h, deg_out, c_out,
             needed_v, src_b, dst_b, w_b, idx_b, val_b, zb, slotmap,
             deg_sh, c_sh, sem):
    cid = lax.axis_index("c")
    sid = lax.axis_index("s")
    wid = cid * NS + sid

    def _z(i, _):
        zb[pl.ds(i * 16, 16)] = jnp.zeros((16,), _f32)
        return 0
    lax.fori_loop(0, (CS_PAD // NS) // 16, _z, 0)
    pltpu.sync_copy(zb, c_sh.at[pl.ds(sid * (CS_PAD // NS), CS_PAD // NS)])
    pltpu.sync_copy(zb.at[pl.ds(0, DEG_PAD // NS)],
                    deg_sh.at[pl.ds(sid * (DEG_PAD // NS), DEG_PAD // NS)])

    pltpu.sync_copy(needed_h, needed_v)
    pltpu.sync_copy(src_h.at[wid], src_b)
    pltpu.sync_copy(dst_h.at[wid], dst_b)
    pltpu.sync_copy(w_h.at[wid], w_b)

    def _zs(i, _):
        slotmap[pl.ds(i * 16, 16)] = jnp.zeros((16,), _i32)
        return 0
    lax.fori_loop(0, P // 16, _zs, 0)
    needed_vec = needed_v[...]
    repv = jnp.full((16,), SLOTS, _i32)
    for s in range(SLOTS):
        ns = needed_vec[s]
        repv = jnp.minimum(repv, jnp.where(needed_vec == ns, s, SLOTS))
    plsc.store_scatter(slotmap, [needed_vec], repv + 1)

    plsc.subcore_barrier()

    def _compute(j, _):
        for k in range(WIN // 16):
            off = k * 16
            srcv = src_b[j, pl.ds(off, 16)]
            dstv = dst_b[j, pl.ds(off, 16)]
            wv = w_b[j, pl.ds(off, 16)]
            slotv = plsc.load_gather(slotmap, [dstv])
            idx = srcv * SLOTS + jnp.maximum(slotv - 1, 0)
            val = jnp.where(slotv > 0, wv, jnp.zeros((16,), _f32))
            idx_b[j, pl.ds(off, 16)] = idx
            val_b[j, pl.ds(off, 16)] = val
        return 0
    lax.fori_loop(0, NWIN, _compute, 0)

    def _chunk(c, _):
        @pl.when(c < NCHUNK)
        def _fire():
            def _f(j, _):
                pltpu.async_copy(w_b.at[j], deg_sh.at[dst_b.at[j]], sem,
                                 add=True)
                pltpu.async_copy(val_b.at[j], c_sh.at[idx_b.at[j]], sem,
                                 add=True)
                return 0
            lax.fori_loop(c * WPC, (c + 1) * WPC, _f, 0)

        @pl.when(c > 0)
        def _drain():
            def _d(j, _):
                pltpu.make_async_copy(w_b.at[j], deg_sh.at[dst_b.at[j]],
                                      sem).wait()
                pltpu.make_async_copy(val_b.at[j], c_sh.at[idx_b.at[j]],
                                      sem).wait()
                return 0
            lax.fori_loop((c - 1) * WPC, c * WPC, _d, 0)
        return 0
    lax.fori_loop(0, NCHUNK + 1, _chunk, 0)

    plsc.subcore_barrier()

    pltpu.sync_copy(deg_sh.at[pl.ds(sid * (DEG_PAD // NS), DEG_PAD // NS)],
                    deg_out.at[cid, 0, pl.ds(sid * (DEG_PAD // NS), DEG_PAD // NS)])
    pltpu.sync_copy(c_sh.at[pl.ds(sid * (CS_PAD // NS), CS_PAD // NS)],
                    c_out.at[cid, 0, pl.ds(sid * (CS_PAD // NS), CS_PAD // NS)])


def _sc_edges(src2, dst2, w2, needed):
    mesh = plsc.VectorSubcoreMesh(core_axis_name="c", subcore_axis_name="s",
                                  num_cores=NC, num_subcores=NS)
    kern = pl.kernel(
        _sc_body,
        out_type=(jax.ShapeDtypeStruct((NC, 1, DEG_PAD), _f32),
                  jax.ShapeDtypeStruct((NC, 1, CS_PAD), _f32)),
        mesh=mesh,
        scratch_types=dict(
            needed_v=pltpu.VMEM((16,), _i32),
            src_b=pltpu.VMEM((NWIN, WIN), _i32),
            dst_b=pltpu.VMEM((NWIN, WIN), _i32),
            w_b=pltpu.VMEM((NWIN, WIN), _f32),
            idx_b=pltpu.VMEM((NWIN, WIN), _i32),
            val_b=pltpu.VMEM((NWIN, WIN), _f32),
            zb=pltpu.VMEM((CS_PAD // NS,), _f32),
            slotmap=pltpu.VMEM((P,), _i32),
            deg_sh=pltpu.VMEM_SHARED((DEG_PAD,), _f32),
            c_sh=pltpu.VMEM_SHARED((CS_PAD,), _f32),
            sem=pltpu.SemaphoreType.DMA,
        ),
        compiler_params=pltpu.CompilerParams(needs_layout_passes=False),
    )
    return kern(src2, dst2, w2, needed)


def _dot_t(a, b, precision=None):
    return lax.dot_general(a, b, (((1,), (1,)), ((), ())),
                           precision=precision,
                           preferred_element_type=_f32)


def _dot_th(a, b):
    return _dot_t(a, b, precision=lax.Precision.HIGHEST)


def _bn_rows(x, g, b):
    mu = jnp.mean(x, axis=0, keepdims=True)
    d = x - mu
    v = jnp.mean(d * d, axis=0, keepdims=True)
    return d * lax.rsqrt(v + EPS) * g + b


GBS = 1000
NGB = G // GBS
NROWS = float(B * G)


def _stage1_body(ge_blk, ge_full, deg_ref, c_ref, pe_ref, needed_ref,
                 bn_emb_g, bn_emb_be, sg_w, sg_b,
                 fw0, fb0, fg0, fbe0, fw1, fb1, fg1, fbe1,
                 bn_pb_g, bn_pb_be, w0, b0,
                 alpha_out, cvec2_out, s1_out, s2_out):
    i = pl.program_id(0)

    @pl.when(i == 0)
    def _():
        deg = deg_ref[0] + deg_ref[1] + 1.0
        dinv = lax.rsqrt(deg + 1e-12)
        c = c_ref[0] + c_ref[1]
        needed = needed_ref[...]
        onehot = (lax.broadcasted_iota(_i32, (P, SLOTS), 0) == needed).astype(_f32)
        dinv_n = jnp.sum(onehot * dinv, axis=0, keepdims=True)
        d = dinv * c * dinv_n + onehot * (dinv_n * dinv_n)
        agg = lax.dot_general(d, pe_ref[...], (((0,), (0,)), ((), ())),
                              precision=lax.Precision.HIGHEST,
                              preferred_element_type=_f32)
        slot_ids = lax.broadcasted_iota(_i32, (SLOTS, SLOTS), 1)
        eq = jnp.transpose(needed) == needed
        rep = jnp.min(jnp.where(eq, slot_ids, SLOTS), axis=1, keepdims=True)
        rmat = (rep == slot_ids).astype(_f32)
        agg_f = jnp.dot(rmat, agg, precision=lax.Precision.HIGHEST,
                        preferred_element_type=_f32)
        asum = jnp.sum(agg_f.reshape(B, 2, H), axis=1)
        pert_sum = _dot_th(asum, sg_w[...]) + 2.0 * sg_b[...]

        t = _dot_th(pert_sum, fw0[...]) + fb0[...]
        t = _bn_rows(t, fg0[...], fbe0[...])
        t = jnp.maximum(t, 0.0)
        t = _dot_th(t, fw1[...]) + fb1[...]
        emb_total = _bn_rows(t, fg1[...], fbe1[...])

        ge = ge_full[...]
        me = jnp.mean(ge, axis=0, keepdims=True)
        ve = jnp.mean(ge * ge, axis=0, keepdims=True) - me * me
        rs = lax.rsqrt(ve + EPS)
        var_embbn = bn_emb_g[...] ** 2 * (ve / (ve + EPS))
        mu_t = jnp.mean(emb_total, axis=0, keepdims=True)
        dt = emb_total - mu_t
        var_t = jnp.mean(dt * dt, axis=0, keepdims=True)
        m_pb = bn_emb_be[...] + mu_t
        t_pb = bn_pb_g[...] * lax.rsqrt(var_embbn + var_t + EPS)
        alpha = rs * bn_emb_g[...] * t_pb
        beta = (bn_emb_be[...] - me * rs * bn_emb_g[...]) * t_pb
        cvec = emb_total * t_pb + bn_pb_be[...] - m_pb * t_pb
        alpha_out[...] = alpha
        cvec2_out[...] = cvec + beta
        s1_out[...] = jnp.zeros_like(s1_out)
        s2_out[...] = jnp.zeros_like(s2_out)

    z = jnp.maximum(ge_blk[...][None, :, :] * alpha_out[...]
                    + cvec2_out[...][:, None, :], 0.0).reshape(B * GBS, H)
    s1_out[...] += jnp.sum(z, axis=0, keepdims=True)
    s2_out[...] += lax.dot_general(z, z, (((0,), (0,)), ((), ())),
                                   preferred_element_type=_f32)


def _stage2_body(ge_blk, alpha, cvec2, w0, b0, g0, be0, s1, s2, w1, b1,
                 y2_out, s1b_out, s2b_out):
    z = jnp.maximum(ge_blk[...][None, :, :] * alpha[...]
                    + cvec2[...][:, None, :], 0.0).reshape(B * GBS, H)
    y1 = _dot_t(z, w0[...]) + b0[...]
    mz = s1[...] / NROWS
    cov = s2[...] / NROWS - jnp.transpose(mz) * mz
    m1 = _dot_t(mz, w0[...]) + b0[...]
    tmp = lax.dot_general(w0[...], cov, (((1,), (0,)), ((), ())),
                          preferred_element_type=_f32)
    v1 = _dot_t(jnp.ones((1, H), _f32), tmp * w0[...])
    t1 = g0[...] * lax.rsqrt(v1 + EPS)
    h = jnp.maximum((y1 - m1) * t1 + be0[...], 0.0)
    y2 = _dot_t(h, w1[...]) + b1[...]
    y2_out[...] = y2.reshape(B, GBS, H)

    @pl.when(pl.program_id(0) == 0)
    def _():
        s1b_out[...] = jnp.zeros_like(s1b_out)
        s2b_out[...] = jnp.zeros_like(s2b_out)
    s1b_out[...] += jnp.sum(y2, axis=0, keepdims=True)
    s2b_out[...] += jnp.sum(y2 * y2, axis=0, keepdims=True)


def _reduce_body(y2_ref, s1b, s2b, g1, be1, v1g_ref, b1col_ref, out1t_ref):
    m2 = s1b[...] / NROWS
    v2 = s2b[...] / NROWS - m2 * m2
    t2 = g1[...] * lax.rsqrt(v2 + EPS)
    v1g = v1g_ref[...]
    vt = v1g * t2
    w = jnp.sum(y2_ref[...] * vt[None, :, :], axis=2)
    dvec = jnp.sum(v1g * (be1[...] - m2 * t2), axis=1, keepdims=True)
    out1t_ref[...] = jnp.transpose(w) + dvec + b1col_ref[...]


def _final_body(out1t_ref, x2_ref, cw0, cb0, cg0, cbe0, cw1, cb1, cg1, cbe1,
                w2a_ref, w2h_ref, b2_ref, out_ref):
    out1 = jnp.transpose(out1t_ref[...])
    c1 = _dot_th(out1, cw0[...]) + cb0[...]
    c1 = _bn_rows(c1, cg0[...], cbe0[...])
    c1 = jnp.maximum(c1, 0.0)
    c1 = _dot_th(c1, cw1[...]) + cb1[...]
    cg = _bn_rows(c1, cg1[...], cbe1[...])
    out_ref[...] = (out1 * w2a_ref[...] + _dot_th(cg, w2h_ref[...])
                    + b2_ref[...] + x2_ref[...])


def _const_spec(shape):
    return pl.BlockSpec(shape, lambda i: tuple(0 for _ in shape))


def kernel(x, pert_idx, edge_index, edge_weight, params):
    p = params
    src2 = edge_index[0].reshape(NW, NWIN, WIN).astype(_i32)
    dst2 = edge_index[1].reshape(NW, NWIN, WIN).astype(_i32)
    w2 = edge_weight.reshape(NW, NWIN, WIN)
    needed = pert_idx.reshape(2 * B).astype(_i32)

    deg2, c2 = _sc_edges(src2, dst2, w2, needed)
    deg2 = deg2[:, 0, :P].reshape(NC, P, 1)
    c2 = c2[:, 0, :P * SLOTS].reshape(NC, P, SLOTS)

    row = lambda a: a.reshape(1, -1)
    ge_spec = pl.BlockSpec((GBS, H), lambda i: (i, 0))
    grid = (NGB,)
    w0, b0 = p['rec_W0'], row(p['rec_b0'])
    g0, be0 = row(p['rec_g0']), row(p['rec_be0'])
    w1, b1 = p['rec_W1'], row(p['rec_b1'])
    g1, be1 = row(p['rec_g1']), row(p['rec_be1'])

    alpha, cvec2, s1, s2 = pl.pallas_call(
        _stage1_body,
        grid=grid,
        in_specs=[ge_spec, _const_spec((G, H)), _const_spec((NC, P, 1)),
                  _const_spec((NC, P, SLOTS)), _const_spec((P, H)),
                  _const_spec((1, 2 * B)),
                  _const_spec((1, H)), _const_spec((1, H)),
                  _const_spec((H, H)), _const_spec((1, H)),
                  _const_spec((H, H)), _const_spec((1, H)),
                  _const_spec((1, H)), _const_spec((1, H)),
                  _const_spec((H, H)), _const_spec((1, H)),
                  _const_spec((1, H)), _const_spec((1, H)),
                  _const_spec((1, H)), _const_spec((1, H)),
                  _const_spec((2 * H, H)), _const_spec((1, 2 * H))],
        out_specs=(_const_spec((1, H)), _const_spec((B, H)),
                   _const_spec((1, H)), _const_spec((H, H))),
        out_shape=(jax.ShapeDtypeStruct((1, H), _f32),
                   jax.ShapeDtypeStruct((B, H), _f32),
                   jax.ShapeDtypeStruct((1, H), _f32),
                   jax.ShapeDtypeStruct((H, H), _f32)),
    )(p['gene_emb'], p['gene_emb'], deg2, c2, p['pert_emb'],
      needed.reshape(1, 2 * B),
      row(p['bn_emb_g']), row(p['bn_emb_be']), p['sg_W'], row(p['sg_b']),
      p['fuse_W0'], row(p['fuse_b0']), row(p['fuse_g0']), row(p['fuse_be0']),
      p['fuse_W1'], row(p['fuse_b1']), row(p['fuse_g1']), row(p['fuse_be1']),
      row(p['bn_pb_g']), row(p['bn_pb_be']), w0, b0)

    y2, s1b, s2b = pl.pallas_call(
        _stage2_body,
        grid=grid,
        in_specs=[ge_spec, _const_spec((1, H)), _const_spec((B, H)),
                  _const_spec((2 * H, H)), _const_spec((1, 2 * H)),
                  _const_spec((1, 2 * H)), _const_spec((1, 2 * H)),
                  _const_spec((1, H)), _const_spec((H, H)),
                  _const_spec((H, 2 * H)), _const_spec((1, H))],
        out_specs=(pl.BlockSpec((B, GBS, H), lambda i: (0, i, 0)),
                   _const_spec((1, H)), _const_spec((1, H))),
        out_shape=(jax.ShapeDtypeStruct((B, G, H), _f32),
                   jax.ShapeDtypeStruct((1, H), _f32),
                   jax.ShapeDtypeStruct((1, H), _f32)),
    )(p['gene_emb'], alpha, cvec2, w0, b0, g0, be0, s1, s2, w1, b1)

    v1g = p['indv_w1'][:, :, 0]
    b1col = p['indv_b1']
    out1t = pl.pallas_call(
        _reduce_body,
        grid=grid,
        in_specs=[pl.BlockSpec((B, GBS, H), lambda i: (0, i, 0)),
                  _const_spec((1, H)), _const_spec((1, H)),
                  _const_spec((1, H)), _const_spec((1, H)),
                  pl.BlockSpec((GBS, H), lambda i: (i, 0)),
                  pl.BlockSpec((GBS, 1), lambda i: (i, 0))],
        out_specs=pl.BlockSpec((GBS, B), lambda i: (i, 0)),
        out_shape=jax.ShapeDtypeStruct((G, B), _f32),
    )(y2, s1b, s2b, g1, be1, v1g, b1col)

    x2 = x.reshape(B, G + 1)[:, :-1]
    w2a = p['indv_w2'][0, :, 0].reshape(1, G)
    w2h = p['indv_w2'][0, :, 1:]
    b2row = p['indv_b2'][0].reshape(1, G)
    final = pl.pallas_call(
        _final_body,
        out_shape=jax.ShapeDtypeStruct((B, G), _f32),
    )(out1t, x2,
      p['cg_W0'], row(p['cg_b0']), row(p['cg_g0']), row(p['cg_be0']),
      p['cg_W1'], row(p['cg_b1']), row(p['cg_g1']), row(p['cg_be1']),
      w2a, w2h, b2row)
    return final


# speedup vs baseline: 2.8112x; 1.0299x over previous
import functools

import jax
import jax.numpy as jnp
from jax import lax
from jax.experimental import pallas as pl
from jax.experimental.pallas import tpu as pltpu
import jax.experimental.pallas.tpu_sc as plsc

G = 5000
P = 10000
H = 128
B = 8
E = 320000
EPS = 1e-5

NC, NS = 2, 16
NW = NC * NS
EPW = E // NW
WIN = 80
NWIN = EPW // WIN
WPC = 5
NCHUNK = NWIN // WPC
DEG_PAD = 10240
SLOTS = 16
CS_PAD = 16 * 10240

_f32 = jnp.float32
_i32 = jnp.int32


def _sc_body(src_h, dst_h, w_h, needed_h, deg_out, c_out,
             needed_v, src_b, dst_b, w_b, idx_b, val_b, zb, slotmap,
             deg_sh, c_sh, sem):
    cid = lax.axis_index("c")
    sid = lax.axis_index("s")
    wid = cid * NS + sid

    def _z(i, _):
        zb[pl.ds(i * 16, 16)] = jnp.zeros((16,), _f32)
        return 0
    lax.fori_loop(0, (CS_PAD // NS) // 16, _z, 0)
    pltpu.sync_copy(zb, c_sh.at[pl.ds(sid * (CS_PAD // NS), CS_PAD // NS)])
    pltpu.sync_copy(zb.at[pl.ds(0, DEG_PAD // NS)],
                    deg_sh.at[pl.ds(sid * (DEG_PAD // NS), DEG_PAD // NS)])

    pltpu.sync_copy(needed_h, needed_v)
    pltpu.sync_copy(src_h.at[wid], src_b)
    pltpu.sync_copy(dst_h.at[wid], dst_b)
    pltpu.sync_copy(w_h.at[wid], w_b)

    def _zs(i, _):
        slotmap[pl.ds(i * 16, 16)] = jnp.zeros((16,), _i32)
        return 0
    lax.fori_loop(0, P // 16, _zs, 0)
    needed_vec = needed_v[...]
    repv = jnp.full((16,), SLOTS, _i32)
    for s in range(SLOTS):
        ns = needed_vec[s]
        repv = jnp.minimum(repv, jnp.where(needed_vec == ns, s, SLOTS))
    plsc.store_scatter(slotmap, [needed_vec], repv + 1)

    plsc.subcore_barrier()

    def _compute(j, _):
        for k in range(WIN // 16):
            off = k * 16
            srcv = src_b[j, pl.ds(off, 16)]
            dstv = dst_b[j, pl.ds(off, 16)]
            wv = w_b[j, pl.ds(off, 16)]
            slotv = plsc.load_gather(slotmap, [dstv])
            idx = srcv * SLOTS + jnp.maximum(slotv - 1, 0)
            val = jnp.where(slotv > 0, wv, jnp.zeros((16,), _f32))
            idx_b[j, pl.ds(off, 16)] = idx
            val_b[j, pl.ds(off, 16)] = val
        return 0
    lax.fori_loop(0, NWIN, _compute, 0)

    def _chunk(c, _):
        @pl.when(c < NCHUNK)
        def _fire():
            def _f(j, _):
                pltpu.async_copy(w_b.at[j], deg_sh.at[dst_b.at[j]], sem,
                                 add=True)
                pltpu.async_copy(val_b.at[j], c_sh.at[idx_b.at[j]], sem,
                                 add=True)
                return 0
            lax.fori_loop(c * WPC, (c + 1) * WPC, _f, 0)

        @pl.when(c > 0)
        def _drain():
            def _d(j, _):
                pltpu.make_async_copy(w_b.at[j], deg_sh.at[dst_b.at[j]],
                                      sem).wait()
                pltpu.make_async_copy(val_b.at[j], c_sh.at[idx_b.at[j]],
                                      sem).wait()
                return 0
            lax.fori_loop((c - 1) * WPC, c * WPC, _d, 0)
        return 0
    lax.fori_loop(0, NCHUNK + 1, _chunk, 0)

    plsc.subcore_barrier()

    pltpu.sync_copy(deg_sh.at[pl.ds(sid * (DEG_PAD // NS), DEG_PAD // NS)],
                    deg_out.at[cid, 0, pl.ds(sid * (DEG_PAD // NS), DEG_PAD // NS)])
    pltpu.sync_copy(c_sh.at[pl.ds(sid * (CS_PAD // NS), CS_PAD // NS)],
                    c_out.at[cid, 0, pl.ds(sid * (CS_PAD // NS), CS_PAD // NS)])


def _sc_edges(src2, dst2, w2, needed):
    mesh = plsc.VectorSubcoreMesh(core_axis_name="c", subcore_axis_name="s",
                                  num_cores=NC, num_subcores=NS)
    kern = pl.kernel(
        _sc_body,
        out_type=(jax.ShapeDtypeStruct((NC, 1, DEG_PAD), _f32),
                  jax.ShapeDtypeStruct((NC, 1, CS_PAD), _f32)),
        mesh=mesh,
        scratch_types=dict(
            needed_v=pltpu.VMEM((16,), _i32),
            src_b=pltpu.VMEM((NWIN, WIN), _i32),
            dst_b=pltpu.VMEM((NWIN, WIN), _i32),
            w_b=pltpu.VMEM((NWIN, WIN), _f32),
            idx_b=pltpu.VMEM((NWIN, WIN), _i32),
            val_b=pltpu.VMEM((NWIN, WIN), _f32),
            zb=pltpu.VMEM((CS_PAD // NS,), _f32),
            slotmap=pltpu.VMEM((P,), _i32),
            deg_sh=pltpu.VMEM_SHARED((DEG_PAD,), _f32),
            c_sh=pltpu.VMEM_SHARED((CS_PAD,), _f32),
            sem=pltpu.SemaphoreType.DMA,
        ),
        compiler_params=pltpu.CompilerParams(needs_layout_passes=False),
    )
    return kern(src2, dst2, w2, needed)


def _dot_t(a, b, precision=None):
    return lax.dot_general(a, b, (((1,), (1,)), ((), ())),
                           precision=precision,
                           preferred_element_type=_f32)


def _dot_th(a, b):
    return _dot_t(a, b, precision=lax.Precision.HIGHEST)


def _bn_rows(x, g, b):
    mu = jnp.mean(x, axis=0, keepdims=True)
    d = x - mu
    v = jnp.mean(d * d, axis=0, keepdims=True)
    return d * lax.rsqrt(v + EPS) * g + b


GBS = 1000
NGB = G // GBS
NROWS = float(B * G)


def _prep_body(ge_full, deg_ref, c_ref, pe_ref, needed_ref,
               bn_emb_g, bn_emb_be, sg_w, sg_b,
               fw0, fb0, fg0, fbe0, fw1, fb1, fg1, fbe1,
               bn_pb_g, bn_pb_be, alpha_out, cvec2_out):
    deg = deg_ref[0] + deg_ref[1] + 1.0
    dinv = lax.rsqrt(deg + 1e-12)
    c = c_ref[0] + c_ref[1]
    needed = needed_ref[...]
    onehot = (lax.broadcasted_iota(_i32, (P, SLOTS), 0) == needed).astype(_f32)
    dinv_n = jnp.sum(onehot * dinv, axis=0, keepdims=True)
    d = dinv * c * dinv_n + onehot * (dinv_n * dinv_n)
    agg = lax.dot_general(d, pe_ref[...], (((0,), (0,)), ((), ())),
                          precision=lax.Precision.HIGHEST,
                          preferred_element_type=_f32)
    slot_ids = lax.broadcasted_iota(_i32, (SLOTS, SLOTS), 1)
    eq = jnp.transpose(needed) == needed
    rep = jnp.min(jnp.where(eq, slot_ids, SLOTS), axis=1, keepdims=True)
    rmat = (rep == slot_ids).astype(_f32)
    agg_f = jnp.dot(rmat, agg, precision=lax.Precision.HIGHEST,
                    preferred_element_type=_f32)
    asum = jnp.sum(agg_f.reshape(B, 2, H), axis=1)
    pert_sum = _dot_th(asum, sg_w[...]) + 2.0 * sg_b[...]

    t = _dot_th(pert_sum, fw0[...]) + fb0[...]
    t = _bn_rows(t, fg0[...], fbe0[...])
    t = jnp.maximum(t, 0.0)
    t = _dot_th(t, fw1[...]) + fb1[...]
    emb_total = _bn_rows(t, fg1[...], fbe1[...])

    ge = ge_full[...]
    me = jnp.mean(ge, axis=0, keepdims=True)
    ve = jnp.mean(ge * ge, axis=0, keepdims=True) - me * me
    rs = lax.rsqrt(ve + EPS)
    var_embbn = bn_emb_g[...] ** 2 * (ve / (ve + EPS))
    mu_t = jnp.mean(emb_total, axis=0, keepdims=True)
    dt = emb_total - mu_t
    var_t = jnp.mean(dt * dt, axis=0, keepdims=True)
    m_pb = bn_emb_be[...] + mu_t
    t_pb = bn_pb_g[...] * lax.rsqrt(var_embbn + var_t + EPS)
    alpha_out[...] = rs * bn_emb_g[...] * t_pb
    beta = (bn_emb_be[...] - me * rs * bn_emb_g[...]) * t_pb
    cvec = emb_total * t_pb + bn_pb_be[...] - m_pb * t_pb
    cvec2_out[...] = cvec + beta


def _tc_body(ge_blk, alpha_ref, cvec2_ref,
             w0, b0, g0, be0, w1, b1, g1, be1,
             v1g_blk, b1col_blk, x2_ref,
             cw0, cb0, cg0, cbe0, cw1, cb1, cg1, cbe1,
             w2a_ref, w2h_ref, b2_ref, out_ref,
             s1s, s2s, s1bs, s2bs, y2s, out1t_s):
    ps = pl.program_id(0)
    i = pl.program_id(1)
    goff = pl.multiple_of(i * GBS, GBS)

    @pl.when((ps == 0) & (i == 0))
    def _():
        s1s[...] = jnp.zeros_like(s1s)
        s2s[...] = jnp.zeros_like(s2s)
        s1bs[...] = jnp.zeros_like(s1bs)
        s2bs[...] = jnp.zeros_like(s2bs)

    def _z():
        return jnp.maximum(ge_blk[...][None, :, :] * alpha_ref[...]
                           + cvec2_ref[...][:, None, :], 0.0).reshape(B * GBS, H)

    @pl.when(ps == 0)
    def _():
        z = _z()
        s1s[...] += jnp.sum(z, axis=0, keepdims=True)
        s2s[...] += lax.dot_general(z, z, (((0,), (0,)), ((), ())),
                                    preferred_element_type=_f32)

    @pl.when(ps == 1)
    def _():
        z = _z()
        y1 = _dot_t(z, w0[...]) + b0[...]
        mz = s1s[...] / NROWS
        cov = s2s[...] / NROWS - jnp.transpose(mz) * mz
        m1 = _dot_t(mz, w0[...]) + b0[...]
        tmp = lax.dot_general(w0[...], cov, (((1,), (0,)), ((), ())),
                              preferred_element_type=_f32)
        v1 = _dot_t(jnp.ones((1, H), _f32), tmp * w0[...])
        t1 = g0[...] * lax.rsqrt(v1 + EPS)
        h = jnp.maximum((y1 - m1) * t1 + be0[...], 0.0)
        y2 = _dot_t(h, w1[...]) + b1[...]
        y2s[:, pl.ds(goff, GBS), :] = y2.reshape(B, GBS, H)
        s1bs[...] += jnp.sum(y2, axis=0, keepdims=True)
        s2bs[...] += jnp.sum(y2 * y2, axis=0, keepdims=True)

    @pl.when(ps == 2)
    def _():
        m2 = s1bs[...] / NROWS
        v2 = s2bs[...] / NROWS - m2 * m2
        t2 = g1[...] * lax.rsqrt(v2 + EPS)
        v1g = v1g_blk[...]
        vt = v1g * t2
        y2 = y2s[:, pl.ds(goff, GBS), :]
        w = jnp.sum(y2 * vt[None, :, :], axis=2)
        dvec = jnp.sum(v1g * (be1[...] - m2 * t2), axis=1, keepdims=True)
        out1t_s[pl.ds(goff, GBS), :] = (jnp.transpose(w) + dvec
                                        + b1col_blk[...])

    @pl.when((ps == 3) & (i == 0))
    def _():
        out1 = jnp.transpose(out1t_s[...])
        c1 = _dot_th(out1, cw0[...]) + cb0[...]
        c1 = _bn_rows(c1, cg0[...], cbe0[...])
        c1 = jnp.maximum(c1, 0.0)
        c1 = _dot_th(c1, cw1[...]) + cb1[...]
        cg = _bn_rows(c1, cg1[...], cbe1[...])
        out_ref[...] = (out1 * w2a_ref[...] + _dot_th(cg, w2h_ref[...])
                        + b2_ref[...] + x2_ref[...])


def _const_spec(shape):
    return pl.BlockSpec(shape, lambda p, i: tuple(0 for _ in shape))


def kernel(x, pert_idx, edge_index, edge_weight, params):
    p = params
    src2 = edge_index[0].reshape(NW, NWIN, WIN).astype(_i32)
    dst2 = edge_index[1].reshape(NW, NWIN, WIN).astype(_i32)
    w2 = edge_weight.reshape(NW, NWIN, WIN)
    needed = pert_idx.reshape(2 * B).astype(_i32)

    deg2, c2 = _sc_edges(src2, dst2, w2, needed)
    deg2 = deg2[:, 0, :P].reshape(NC, P, 1)
    c2 = c2[:, 0, :P * SLOTS].reshape(NC, P, SLOTS)

    row = lambda a: a.reshape(1, -1)
    x2 = x.reshape(B, G + 1)[:, :-1]

    alpha, cvec2 = pl.pallas_call(
        _prep_body,
        out_shape=(jax.ShapeDtypeStruct((1, H), _f32),
                   jax.ShapeDtypeStruct((B, H), _f32)),
    )(p['gene_emb'], deg2, c2, p['pert_emb'], needed.reshape(1, 2 * B),
      row(p['bn_emb_g']), row(p['bn_emb_be']), p['sg_W'], row(p['sg_b']),
      p['fuse_W0'], row(p['fuse_b0']), row(p['fuse_g0']), row(p['fuse_be0']),
      p['fuse_W1'], row(p['fuse_b1']), row(p['fuse_g1']), row(p['fuse_be1']),
      row(p['bn_pb_g']), row(p['bn_pb_be']))

    final = pl.pallas_call(
        _tc_body,
        grid=(4, NGB),
        in_specs=[pl.BlockSpec((GBS, H), lambda p, i: (i, 0)),
                  _const_spec((1, H)), _const_spec((B, H)),
                  _const_spec((2 * H, H)), _const_spec((1, 2 * H)),
                  _const_spec((1, 2 * H)), _const_spec((1, 2 * H)),
                  _const_spec((H, 2 * H)), _const_spec((1, H)),
                  _const_spec((1, H)), _const_spec((1, H)),
                  pl.BlockSpec((GBS, H), lambda p, i: (i, 0)),
                  pl.BlockSpec((GBS, 1), lambda p, i: (i, 0)),
                  _const_spec((B, G)),
                  _const_spec((H, G)), _const_spec((1, H)),
                  _const_spec((1, H)), _const_spec((1, H)),
                  _const_spec((H, H)), _const_spec((1, H)),
                  _const_spec((1, H)), _const_spec((1, H)),
                  _const_spec((1, G)), _const_spec((G, H)),
                  _const_spec((1, G))],
        out_specs=_const_spec((B, G)),
        out_shape=jax.ShapeDtypeStruct((B, G), _f32),
        scratch_shapes=[pltpu.VMEM((1, H), _f32), pltpu.VMEM((H, H), _f32),
                        pltpu.VMEM((1, H), _f32), pltpu.VMEM((1, H), _f32),
                        pltpu.VMEM((B, G, H), _f32), pltpu.VMEM((G, B), _f32)],
    )(p['gene_emb'], alpha, cvec2,
      p['rec_W0'], row(p['rec_b0']), row(p['rec_g0']), row(p['rec_be0']),
      p['rec_W1'], row(p['rec_b1']), row(p['rec_g1']), row(p['rec_be1']),
      p['indv_w1'][:, :, 0], p['indv_b1'], x2,
      p['cg_W0'], row(p['cg_b0']), row(p['cg_g0']), row(p['cg_be0']),
      p['cg_W1'], row(p['cg_b1']), row(p['cg_g1']), row(p['cg_be1']),
      p['indv_w2'][0, :, 0].reshape(1, G), p['indv_w2'][0, :, 1:],
      p['indv_b2'][0].reshape(1, G))
    return final


# speedup vs baseline: 2.9932x; 1.0647x over previous
import functools

import jax
import jax.numpy as jnp
from jax import lax
from jax.experimental import pallas as pl
from jax.experimental.pallas import tpu as pltpu
import jax.experimental.pallas.tpu_sc as plsc

G = 5000
P = 10000
H = 128
B = 8
E = 320000
EPS = 1e-5

NC, NS = 2, 16
NW = NC * NS
EPW = E // NW
WIN = 80
NWIN = EPW // WIN
WPC = 5
NCHUNK = NWIN // WPC
DEG_PAD = 10240
SLOTS = 16
CS_PAD = 16 * 10240

_f32 = jnp.float32
_i32 = jnp.int32


def _sc_body(src_h, dst_h, w_h, needed_h, deg_out, c_out,
             needed_v, src_b, dst_b, w_b, idx_b, val_b, zb, slotmap,
             deg_sh, c_sh, sem):
    cid = lax.axis_index("c")
    sid = lax.axis_index("s")
    wid = cid * NS + sid

    def _z(i, _):
        for u in range(5):
            zb[pl.ds(i * 80 + u * 16, 16)] = jnp.zeros((16,), _f32)
        return 0
    lax.fori_loop(0, (CS_PAD // NS) // 80, _z, 0)
    pltpu.sync_copy(zb, c_sh.at[pl.ds(sid * (CS_PAD // NS), CS_PAD // NS)])
    pltpu.sync_copy(zb.at[pl.ds(0, DEG_PAD // NS)],
                    deg_sh.at[pl.ds(sid * (DEG_PAD // NS), DEG_PAD // NS)])

    pltpu.sync_copy(needed_h, needed_v)
    pltpu.sync_copy(src_h.at[wid], src_b)
    pltpu.sync_copy(dst_h.at[wid], dst_b)
    pltpu.sync_copy(w_h.at[wid], w_b)

    def _zs(i, _):
        for u in range(5):
            slotmap[pl.ds(i * 80 + u * 16, 16)] = jnp.zeros((16,), _i32)
        return 0
    lax.fori_loop(0, P // 80, _zs, 0)
    needed_vec = needed_v[...]
    repv = jnp.full((16,), SLOTS, _i32)
    for s in range(SLOTS):
        ns = needed_vec[s]
        repv = jnp.minimum(repv, jnp.where(needed_vec == ns, s, SLOTS))
    plsc.store_scatter(slotmap, [needed_vec], repv + 1)

    plsc.subcore_barrier()

    def _compute(j, _):
        for k in range(WIN // 16):
            off = k * 16
            srcv = src_b[j, pl.ds(off, 16)]
            dstv = dst_b[j, pl.ds(off, 16)]
            wv = w_b[j, pl.ds(off, 16)]
            slotv = plsc.load_gather(slotmap, [dstv])
            idx = srcv * SLOTS + jnp.maximum(slotv - 1, 0)
            val = jnp.where(slotv > 0, wv, jnp.zeros((16,), _f32))
            idx_b[j, pl.ds(off, 16)] = idx
            val_b[j, pl.ds(off, 16)] = val
        pltpu.async_copy(w_b.at[j], deg_sh.at[dst_b.at[j]], sem, add=True)
        pltpu.async_copy(val_b.at[j], c_sh.at[idx_b.at[j]], sem, add=True)
        return 0
    lax.fori_loop(0, NWIN, _compute, 0)

    def _drain(j, _):
        pltpu.make_async_copy(w_b.at[j], deg_sh.at[dst_b.at[j]], sem).wait()
        pltpu.make_async_copy(val_b.at[j], c_sh.at[idx_b.at[j]], sem).wait()
        return 0
    lax.fori_loop(0, NWIN, _drain, 0)

    plsc.subcore_barrier()

    pltpu.sync_copy(deg_sh.at[pl.ds(sid * (DEG_PAD // NS), DEG_PAD // NS)],
                    deg_out.at[cid, 0, pl.ds(sid * (DEG_PAD // NS), DEG_PAD // NS)])
    pltpu.sync_copy(c_sh.at[pl.ds(sid * (CS_PAD // NS), CS_PAD // NS)],
                    c_out.at[cid, 0, pl.ds(sid * (CS_PAD // NS), CS_PAD // NS)])


def _sc_edges(src2, dst2, w2, needed):
    mesh = plsc.VectorSubcoreMesh(core_axis_name="c", subcore_axis_name="s",
                                  num_cores=NC, num_subcores=NS)
    kern = pl.kernel(
        _sc_body,
        out_type=(jax.ShapeDtypeStruct((NC, 1, DEG_PAD), _f32),
                  jax.ShapeDtypeStruct((NC, 1, CS_PAD), _f32)),
        mesh=mesh,
        scratch_types=dict(
            needed_v=pltpu.VMEM((16,), _i32),
            src_b=pltpu.VMEM((NWIN, WIN), _i32),
            dst_b=pltpu.VMEM((NWIN, WIN), _i32),
            w_b=pltpu.VMEM((NWIN, WIN), _f32),
            idx_b=pltpu.VMEM((NWIN, WIN), _i32),
            val_b=pltpu.VMEM((NWIN, WIN), _f32),
            zb=pltpu.VMEM((CS_PAD // NS,), _f32),
            slotmap=pltpu.VMEM((P,), _i32),
            deg_sh=pltpu.VMEM_SHARED((DEG_PAD,), _f32),
            c_sh=pltpu.VMEM_SHARED((CS_PAD,), _f32),
            sem=pltpu.SemaphoreType.DMA,
        ),
        compiler_params=pltpu.CompilerParams(needs_layout_passes=False),
    )
    return kern(src2, dst2, w2, needed)


def _dot_t(a, b, precision=None):
    return lax.dot_general(a, b, (((1,), (1,)), ((), ())),
                           precision=precision,
                           preferred_element_type=_f32)


def _dot_th(a, b):
    return _dot_t(a, b, precision=lax.Precision.HIGHEST)


def _bn_rows(x, g, b):
    mu = jnp.mean(x, axis=0, keepdims=True)
    d = x - mu
    v = jnp.mean(d * d, axis=0, keepdims=True)
    return d * lax.rsqrt(v + EPS) * g + b


GBS = 1000
NGB = G // GBS
NROWS = float(B * G)


def _prep_body(ge_full, deg_ref, c_ref, pe_ref, needed_ref,
               bn_emb_g, bn_emb_be, sg_w, sg_b,
               fw0, fb0, fg0, fbe0, fw1, fb1, fg1, fbe1,
               bn_pb_g, bn_pb_be, alpha_out, cvec2_out):
    deg = deg_ref[0] + deg_ref[1] + 1.0
    dinv = lax.rsqrt(deg + 1e-12)
    c = c_ref[0] + c_ref[1]
    needed = needed_ref[...]
    onehot = (lax.broadcasted_iota(_i32, (P, SLOTS), 0) == needed).astype(_f32)
    dinv_n = jnp.sum(onehot * dinv, axis=0, keepdims=True)
    d = dinv * c * dinv_n + onehot * (dinv_n * dinv_n)
    agg = lax.dot_general(d, pe_ref[...], (((0,), (0,)), ((), ())),
                          precision=lax.Precision.HIGHEST,
                          preferred_element_type=_f32)
    slot_ids = lax.broadcasted_iota(_i32, (SLOTS, SLOTS), 1)
    eq = jnp.transpose(needed) == needed
    rep = jnp.min(jnp.where(eq, slot_ids, SLOTS), axis=1, keepdims=True)
    rmat = (rep == slot_ids).astype(_f32)
    agg_f = jnp.dot(rmat, agg, precision=lax.Precision.HIGHEST,
                    preferred_element_type=_f32)
    asum = jnp.sum(agg_f.reshape(B, 2, H), axis=1)
    pert_sum = _dot_th(asum, sg_w[...]) + 2.0 * sg_b[...]

    t = _dot_th(pert_sum, fw0[...]) + fb0[...]
    t = _bn_rows(t, fg0[...], fbe0[...])
    t = jnp.maximum(t, 0.0)
    t = _dot_th(t, fw1[...]) + fb1[...]
    emb_total = _bn_rows(t, fg1[...], fbe1[...])

    ge = ge_full[...]
    me = jnp.mean(ge, axis=0, keepdims=True)
    ve = jnp.mean(ge * ge, axis=0, keepdims=True) - me * me
    rs = lax.rsqrt(ve + EPS)
    var_embbn = bn_emb_g[...] ** 2 * (ve / (ve + EPS))
    mu_t = jnp.mean(emb_total, axis=0, keepdims=True)
    dt = emb_total - mu_t
    var_t = jnp.mean(dt * dt, axis=0, keepdims=True)
    m_pb = bn_emb_be[...] + mu_t
    t_pb = bn_pb_g[...] * lax.rsqrt(var_embbn + var_t + EPS)
    alpha_out[...] = rs * bn_emb_g[...] * t_pb
    beta = (bn_emb_be[...] - me * rs * bn_emb_g[...]) * t_pb
    cvec = emb_total * t_pb + bn_pb_be[...] - m_pb * t_pb
    cvec2_out[...] = cvec + beta


def _tc_body(ge_blk, alpha_ref, cvec2_ref,
             w0, b0, g0, be0, w1, b1, g1, be1,
             v1g_blk, b1col_blk, x2_ref,
             cw0, cb0, cg0, cbe0, cw1, cb1, cg1, cbe1,
             w2a_ref, w2h_ref, b2_ref, out_ref,
             s1s, s2s, s1bs, s2bs, y2s, out1t_s):
    ps = pl.program_id(0)
    i = pl.program_id(1)
    goff = pl.multiple_of(i * GBS, GBS)

    @pl.when((ps == 0) & (i == 0))
    def _():
        s1s[...] = jnp.zeros_like(s1s)
        s2s[...] = jnp.zeros_like(s2s)
        s1bs[...] = jnp.zeros_like(s1bs)
        s2bs[...] = jnp.zeros_like(s2bs)

    def _z():
        return jnp.maximum(ge_blk[...][None, :, :] * alpha_ref[...]
                           + cvec2_ref[...][:, None, :], 0.0).reshape(B * GBS, H)

    @pl.when(ps == 0)
    def _():
        z = _z()
        s1s[...] += jnp.sum(z, axis=0, keepdims=True)
        s2s[...] += lax.dot_general(z, z, (((0,), (0,)), ((), ())),
                                    preferred_element_type=_f32)

    @pl.when(ps == 1)
    def _():
        z = _z()
        y1 = _dot_t(z, w0[...]) + b0[...]
        mz = s1s[...] / NROWS
        cov = s2s[...] / NROWS - jnp.transpose(mz) * mz
        m1 = _dot_t(mz, w0[...]) + b0[...]
        tmp = lax.dot_general(w0[...], cov, (((1,), (0,)), ((), ())),
                              preferred_element_type=_f32)
        v1 = _dot_t(jnp.ones((1, H), _f32), tmp * w0[...])
        t1 = g0[...] * lax.rsqrt(v1 + EPS)
        h = jnp.maximum((y1 - m1) * t1 + be0[...], 0.0)
        y2 = _dot_t(h, w1[...]) + b1[...]
        y2s[:, pl.ds(goff, GBS), :] = y2.reshape(B, GBS, H)
        s1bs[...] += jnp.sum(y2, axis=0, keepdims=True)
        s2bs[...] += jnp.sum(y2 * y2, axis=0, keepdims=True)

    @pl.when(ps == 2)
    def _():
        m2 = s1bs[...] / NROWS
        v2 = s2bs[...] / NROWS - m2 * m2
        t2 = g1[...] * lax.rsqrt(v2 + EPS)
        v1g = v1g_blk[...]
        vt = v1g * t2
        y2 = y2s[:, pl.ds(goff, GBS), :]
        w = jnp.sum(y2 * vt[None, :, :], axis=2)
        dvec = jnp.sum(v1g * (be1[...] - m2 * t2), axis=1, keepdims=True)
        out1t_s[pl.ds(goff, GBS), :] = (jnp.transpose(w) + dvec
                                        + b1col_blk[...])

    @pl.when((ps == 3) & (i == 0))
    def _():
        out1 = jnp.transpose(out1t_s[...])
        c1 = _dot_th(out1, cw0[...]) + cb0[...]
        c1 = _bn_rows(c1, cg0[...], cbe0[...])
        c1 = jnp.maximum(c1, 0.0)
        c1 = _dot_th(c1, cw1[...]) + cb1[...]
        cg = _bn_rows(c1, cg1[...], cbe1[...])
        out_ref[...] = (out1 * w2a_ref[...] + _dot_th(cg, w2h_ref[...])
                        + b2_ref[...] + x2_ref[...])


def _const_spec(shape):
    return pl.BlockSpec(shape, lambda p, i: tuple(0 for _ in shape))


def kernel(x, pert_idx, edge_index, edge_weight, params):
    p = params
    src2 = edge_index[0].reshape(NW, NWIN, WIN).astype(_i32)
    dst2 = edge_index[1].reshape(NW, NWIN, WIN).astype(_i32)
    w2 = edge_weight.reshape(NW, NWIN, WIN)
    needed = pert_idx.reshape(2 * B).astype(_i32)

    deg2, c2 = _sc_edges(src2, dst2, w2, needed)
    deg2 = deg2[:, 0, :P].reshape(NC, P, 1)
    c2 = c2[:, 0, :P * SLOTS].reshape(NC, P, SLOTS)

    row = lambda a: a.reshape(1, -1)
    x2 = x.reshape(B, G + 1)[:, :-1]

    alpha, cvec2 = pl.pallas_call(
        _prep_body,
        out_shape=(jax.ShapeDtypeStruct((1, H), _f32),
                   jax.ShapeDtypeStruct((B, H), _f32)),
    )(p['gene_emb'], deg2, c2, p['pert_emb'], needed.reshape(1, 2 * B),
      row(p['bn_emb_g']), row(p['bn_emb_be']), p['sg_W'], row(p['sg_b']),
      p['fuse_W0'], row(p['fuse_b0']), row(p['fuse_g0']), row(p['fuse_be0']),
      p['fuse_W1'], row(p['fuse_b1']), row(p['fuse_g1']), row(p['fuse_be1']),
      row(p['bn_pb_g']), row(p['bn_pb_be']))

    final = pl.pallas_call(
        _tc_body,
        grid=(4, NGB),
        in_specs=[pl.BlockSpec((GBS, H), lambda p, i: (i, 0)),
                  _const_spec((1, H)), _const_spec((B, H)),
                  _const_spec((2 * H, H)), _const_spec((1, 2 * H)),
                  _const_spec((1, 2 * H)), _const_spec((1, 2 * H)),
                  _const_spec((H, 2 * H)), _const_spec((1, H)),
                  _const_spec((1, H)), _const_spec((1, H)),
                  pl.BlockSpec((GBS, H), lambda p, i: (i, 0)),
                  pl.BlockSpec((GBS, 1), lambda p, i: (i, 0)),
                  _const_spec((B, G)),
                  _const_spec((H, G)), _const_spec((1, H)),
                  _const_spec((1, H)), _const_spec((1, H)),
                  _const_spec((H, H)), _const_spec((1, H)),
                  _const_spec((1, H)), _const_spec((1, H)),
                  _const_spec((1, G)), _const_spec((G, H)),
                  _const_spec((1, G))],
        out_specs=_const_spec((B, G)),
        out_shape=jax.ShapeDtypeStruct((B, G), _f32),
        scratch_shapes=[pltpu.VMEM((1, H), _f32), pltpu.VMEM((H, H), _f32),
                        pltpu.VMEM((1, H), _f32), pltpu.VMEM((1, H), _f32),
                        pltpu.VMEM((B, G, H), _f32), pltpu.VMEM((G, B), _f32)],
    )(p['gene_emb'], alpha, cvec2,
      p['rec_W0'], row(p['rec_b0']), row(p['rec_g0']), row(p['rec_be0']),
      p['rec_W1'], row(p['rec_b1']), row(p['rec_g1']), row(p['rec_be1']),
      p['indv_w1'][:, :, 0], p['indv_b1'], x2,
      p['cg_W0'], row(p['cg_b0']), row(p['cg_g0']), row(p['cg_be0']),
      p['cg_W1'], row(p['cg_b1']), row(p['cg_g1']), row(p['cg_be1']),
      p['indv_w2'][0, :, 0].reshape(1, G), p['indv_w2'][0, :, 1:],
      p['indv_b2'][0].reshape(1, G))
    return final


# speedup vs baseline: 2.9988x; 1.0019x over previous
import jax
import jax.numpy as jnp
from jax import lax
from jax.experimental import pallas as pl
from jax.experimental.pallas import tpu as pltpu
import jax.experimental.pallas.tpu_sc as plsc

G = 5000
P = 10000
H = 128
B = 8
E = 320000
EPS = 1e-5

NC, NS = 2, 16
NW = NC * NS
EPW = E // NW
WIN = 80
NWIN = EPW // WIN
DEG_PAD = 10240
SLOTS = 16
CS_PAD = 16 * 10240

_f32 = jnp.float32
_i32 = jnp.int32


def _sc_body(src_h, dst_h, w_h, needed_h, deg_out, c_out,
             needed_v, src_b, dst_b, w_b, idx_b, val_b, zb, slotmap,
             deg_sh, c_sh, sem):
    cid = lax.axis_index("c")
    sid = lax.axis_index("s")
    wid = cid * NS + sid

    def _z(i, _):
        for u in range(5):
            zb[pl.ds(i * 80 + u * 16, 16)] = jnp.zeros((16,), _f32)
        return 0
    lax.fori_loop(0, (CS_PAD // NS) // 80, _z, 0)
    pltpu.sync_copy(zb, c_sh.at[pl.ds(sid * (CS_PAD // NS), CS_PAD // NS)])
    pltpu.sync_copy(zb.at[pl.ds(0, DEG_PAD // NS)],
                    deg_sh.at[pl.ds(sid * (DEG_PAD // NS), DEG_PAD // NS)])

    pltpu.sync_copy(needed_h, needed_v)
    pltpu.sync_copy(src_h.at[wid], src_b)
    pltpu.sync_copy(dst_h.at[wid], dst_b)
    pltpu.sync_copy(w_h.at[wid], w_b)

    def _zs(i, _):
        for u in range(5):
            slotmap[pl.ds(i * 80 + u * 16, 16)] = jnp.zeros((16,), _i32)
        return 0
    lax.fori_loop(0, P // 80, _zs, 0)
    needed_vec = needed_v[...]
    repv = jnp.full((16,), SLOTS, _i32)
    for s in range(SLOTS):
        ns = needed_vec[s]
        repv = jnp.minimum(repv, jnp.where(needed_vec == ns, s, SLOTS))
    plsc.store_scatter(slotmap, [needed_vec], repv + 1)

    plsc.subcore_barrier()

    def _compute(j, _):
        for k in range(WIN // 16):
            off = k * 16
            srcv = src_b[j, pl.ds(off, 16)]
            dstv = dst_b[j, pl.ds(off, 16)]
            wv = w_b[j, pl.ds(off, 16)]
            slotv = plsc.load_gather(slotmap, [dstv])
            idx = srcv * SLOTS + jnp.maximum(slotv - 1, 0)
            val = jnp.where(slotv > 0, wv, jnp.zeros((16,), _f32))
            idx_b[j, pl.ds(off, 16)] = idx
            val_b[j, pl.ds(off, 16)] = val
        pltpu.async_copy(w_b.at[j], deg_sh.at[dst_b.at[j]], sem, add=True)
        pltpu.async_copy(val_b.at[j], c_sh.at[idx_b.at[j]], sem, add=True)
        return 0
    lax.fori_loop(0, NWIN, _compute, 0)

    def _drain(j, _):
        pltpu.make_async_copy(w_b.at[j], deg_sh.at[dst_b.at[j]], sem).wait()
        pltpu.make_async_copy(val_b.at[j], c_sh.at[idx_b.at[j]], sem).wait()
        return 0
    lax.fori_loop(0, NWIN, _drain, 0)

    plsc.subcore_barrier()

    pltpu.sync_copy(deg_sh.at[pl.ds(sid * (DEG_PAD // NS), DEG_PAD // NS)],
                    deg_out.at[cid, 0, pl.ds(sid * (DEG_PAD // NS), DEG_PAD // NS)])
    pltpu.sync_copy(c_sh.at[pl.ds(sid * (CS_PAD // NS), CS_PAD // NS)],
                    c_out.at[cid, 0, pl.ds(sid * (CS_PAD // NS), CS_PAD // NS)])


def _sc_edges(src2, dst2, w2, needed):
    mesh = plsc.VectorSubcoreMesh(core_axis_name="c", subcore_axis_name="s",
                                  num_cores=NC, num_subcores=NS)
    kern = pl.kernel(
        _sc_body,
        out_type=(jax.ShapeDtypeStruct((NC, 1, DEG_PAD), _f32),
                  jax.ShapeDtypeStruct((NC, 1, CS_PAD), _f32)),
        mesh=mesh,
        scratch_types=dict(
            needed_v=pltpu.VMEM((16,), _i32),
            src_b=pltpu.VMEM((NWIN, WIN), _i32),
            dst_b=pltpu.VMEM((NWIN, WIN), _i32),
            w_b=pltpu.VMEM((NWIN, WIN), _f32),
            idx_b=pltpu.VMEM((NWIN, WIN), _i32),
            val_b=pltpu.VMEM((NWIN, WIN), _f32),
            zb=pltpu.VMEM((CS_PAD // NS,), _f32),
            slotmap=pltpu.VMEM((P,), _i32),
            deg_sh=pltpu.VMEM_SHARED((DEG_PAD,), _f32),
            c_sh=pltpu.VMEM_SHARED((CS_PAD,), _f32),
            sem=pltpu.SemaphoreType.DMA,
        ),
        compiler_params=pltpu.CompilerParams(needs_layout_passes=False),
    )
    return kern(src2, dst2, w2, needed)


def _dot_t(a, b, precision=None):
    return lax.dot_general(a, b, (((1,), (1,)), ((), ())),
                           precision=precision,
                           preferred_element_type=_f32)


def _dot_th(a, b):
    return _dot_t(a, b, precision=lax.Precision.HIGHEST)


def _bn_rows(x, g, b):
    mu = jnp.mean(x, axis=0, keepdims=True)
    d = x - mu
    v = jnp.mean(d * d, axis=0, keepdims=True)
    return d * lax.rsqrt(v + EPS) * g + b


GBS = 1000
NGB = G // GBS
NROWS = float(B * G)


def _prep_body(ge_full, deg_ref, c_ref, pe_ref, needed_ref,
               bn_emb_g, bn_emb_be, sg_w, sg_b,
               fw0, fb0, fg0, fbe0, fw1, fb1, fg1, fbe1,
               bn_pb_g, bn_pb_be, alpha_out, cvec2_out):
    deg = deg_ref[0] + deg_ref[1] + 1.0
    dinv = lax.rsqrt(deg + 1e-12)
    c = c_ref[0] + c_ref[1]
    needed = needed_ref[...]
    onehot = (lax.broadcasted_iota(_i32, (P, SLOTS), 0) == needed).astype(_f32)
    dinv_n = jnp.sum(onehot * dinv, axis=0, keepdims=True)
    d = dinv * c * dinv_n + onehot * (dinv_n * dinv_n)
    agg = lax.dot_general(d, pe_ref[...], (((0,), (0,)), ((), ())),
                          precision=lax.Precision.HIGHEST,
                          preferred_element_type=_f32)
    slot_ids = lax.broadcasted_iota(_i32, (SLOTS, SLOTS), 1)
    eq = jnp.transpose(needed) == needed
    rep = jnp.min(jnp.where(eq, slot_ids, SLOTS), axis=1, keepdims=True)
    rmat = (rep == slot_ids).astype(_f32)
    agg_f = jnp.dot(rmat, agg, precision=lax.Precision.HIGHEST,
                    preferred_element_type=_f32)
    asum = jnp.sum(agg_f.reshape(B, 2, H), axis=1)
    pert_sum = _dot_th(asum, sg_w[...]) + 2.0 * sg_b[...]

    t = _dot_th(pert_sum, fw0[...]) + fb0[...]
    t = _bn_rows(t, fg0[...], fbe0[...])
    t = jnp.maximum(t, 0.0)
    t = _dot_th(t, fw1[...]) + fb1[...]
    emb_total = _bn_rows(t, fg1[...], fbe1[...])

    ge = ge_full[...]
    me = jnp.mean(ge, axis=0, keepdims=True)
    ve = jnp.mean(ge * ge, axis=0, keepdims=True) - me * me
    rs = lax.rsqrt(ve + EPS)
    var_embbn = bn_emb_g[...] ** 2 * (ve / (ve + EPS))
    mu_t = jnp.mean(emb_total, axis=0, keepdims=True)
    dt = emb_total - mu_t
    var_t = jnp.mean(dt * dt, axis=0, keepdims=True)
    m_pb = bn_emb_be[...] + mu_t
    t_pb = bn_pb_g[...] * lax.rsqrt(var_embbn + var_t + EPS)
    alpha_out[...] = rs * bn_emb_g[...] * t_pb
    beta = (bn_emb_be[...] - me * rs * bn_emb_g[...]) * t_pb
    cvec = emb_total * t_pb + bn_pb_be[...] - m_pb * t_pb
    cvec2_out[...] = cvec + beta


def _tc_body(ge_blk, alpha_ref, cvec2_ref,
             w0, b0, g0, be0, w1, b1, g1, be1,
             v1g_blk, b1col_blk, x2_ref,
             cw0, cb0, cg0, cbe0, cw1, cb1, cg1, cbe1,
             w2a_ref, w2h_ref, b2_ref, out_ref,
             s1s, s2s, s1bs, s2bs, y2s, out1t_s):
    ps = pl.program_id(0)
    i = pl.program_id(1)
    goff = pl.multiple_of(i * GBS, GBS)

    @pl.when((ps == 0) & (i == 0))
    def _():
        s1s[...] = jnp.zeros_like(s1s)
        s2s[...] = jnp.zeros_like(s2s)
        s1bs[...] = jnp.zeros_like(s1bs)
        s2bs[...] = jnp.zeros_like(s2bs)

    def _z():
        return jnp.maximum(ge_blk[...][None, :, :] * alpha_ref[...]
                           + cvec2_ref[...][:, None, :], 0.0).reshape(B * GBS, H)

    @pl.when(ps == 0)
    def _():
        z = _z()
        s1s[...] += jnp.sum(z, axis=0, keepdims=True)
        s2s[...] += lax.dot_general(z, z, (((0,), (0,)), ((), ())),
                                    preferred_element_type=_f32)

    @pl.when(ps == 1)
    def _():
        z = _z()
        y1 = _dot_t(z, w0[...]) + b0[...]
        mz = s1s[...] / NROWS
        cov = s2s[...] / NROWS - jnp.transpose(mz) * mz
        m1 = _dot_t(mz, w0[...]) + b0[...]
        tmp = lax.dot_general(w0[...], cov, (((1,), (0,)), ((), ())),
                              preferred_element_type=_f32)
        v1 = _dot_t(jnp.ones((1, H), _f32), tmp * w0[...])
        t1 = g0[...] * lax.rsqrt(v1 + EPS)
        h = jnp.maximum((y1 - m1) * t1 + be0[...], 0.0)
        y2 = _dot_t(h, w1[...]) + b1[...]
        y2s[:, pl.ds(goff, GBS), :] = y2.reshape(B, GBS, H)
        s1bs[...] += jnp.sum(y2, axis=0, keepdims=True)
        s2bs[...] += jnp.sum(y2 * y2, axis=0, keepdims=True)

    @pl.when(ps == 2)
    def _():
        m2 = s1bs[...] / NROWS
        v2 = s2bs[...] / NROWS - m2 * m2
        t2 = g1[...] * lax.rsqrt(v2 + EPS)
        v1g = v1g_blk[...]
        vt = v1g * t2
        y2 = y2s[:, pl.ds(goff, GBS), :]
        w = jnp.sum(y2 * vt[None, :, :], axis=2)
        dvec = jnp.sum(v1g * (be1[...] - m2 * t2), axis=1, keepdims=True)
        out1t_s[pl.ds(goff, GBS), :] = (jnp.transpose(w) + dvec
                                        + b1col_blk[...])

    @pl.when((ps == 3) & (i == 0))
    def _():
        out1 = jnp.transpose(out1t_s[...])
        c1 = _dot_th(out1, cw0[...]) + cb0[...]
        c1 = _bn_rows(c1, cg0[...], cbe0[...])
        c1 = jnp.maximum(c1, 0.0)
        c1 = _dot_th(c1, cw1[...]) + cb1[...]
        cg = _bn_rows(c1, cg1[...], cbe1[...])
        out_ref[...] = (out1 * w2a_ref[...] + _dot_th(cg, w2h_ref[...])
                        + b2_ref[...] + x2_ref[...])


def _const_spec(shape):
    return pl.BlockSpec(shape, lambda p, i: tuple(0 for _ in shape))


def kernel(x, pert_idx, edge_index, edge_weight, params):
    p = params
    src2 = edge_index[0].reshape(NW, NWIN, WIN).astype(_i32)
    dst2 = edge_index[1].reshape(NW, NWIN, WIN).astype(_i32)
    w2 = edge_weight.reshape(NW, NWIN, WIN)
    needed = pert_idx.reshape(2 * B).astype(_i32)

    deg2, c2 = _sc_edges(src2, dst2, w2, needed)
    deg2 = deg2[:, 0, :P].reshape(NC, P, 1)
    c2 = c2[:, 0, :P * SLOTS].reshape(NC, P, SLOTS)

    row = lambda a: a.reshape(1, -1)
    x2 = x.reshape(B, G + 1)[:, :-1]

    alpha, cvec2 = pl.pallas_call(
        _prep_body,
        out_shape=(jax.ShapeDtypeStruct((1, H), _f32),
                   jax.ShapeDtypeStruct((B, H), _f32)),
    )(p['gene_emb'], deg2, c2, p['pert_emb'], needed.reshape(1, 2 * B),
      row(p['bn_emb_g']), row(p['bn_emb_be']), p['sg_W'], row(p['sg_b']),
      p['fuse_W0'], row(p['fuse_b0']), row(p['fuse_g0']), row(p['fuse_be0']),
      p['fuse_W1'], row(p['fuse_b1']), row(p['fuse_g1']), row(p['fuse_be1']),
      row(p['bn_pb_g']), row(p['bn_pb_be']))

    final = pl.pallas_call(
        _tc_body,
        grid=(4, NGB),
        in_specs=[pl.BlockSpec((GBS, H), lambda p, i: (i, 0)),
                  _const_spec((1, H)), _const_spec((B, H)),
                  _const_spec((2 * H, H)), _const_spec((1, 2 * H)),
                  _const_spec((1, 2 * H)), _const_spec((1, 2 * H)),
                  _const_spec((H, 2 * H)), _const_spec((1, H)),
                  _const_spec((1, H)), _const_spec((1, H)),
                  pl.BlockSpec((GBS, H), lambda p, i: (i, 0)),
                  pl.BlockSpec((GBS, 1), lambda p, i: (i, 0)),
                  _const_spec((B, G)),
                  _const_spec((H, G)), _const_spec((1, H)),
                  _const_spec((1, H)), _const_spec((1, H)),
                  _const_spec((H, H)), _const_spec((1, H)),
                  _const_spec((1, H)), _const_spec((1, H)),
                  _const_spec((1, G)), _const_spec((G, H)),
                  _const_spec((1, G))],
        out_specs=_const_spec((B, G)),
        out_shape=jax.ShapeDtypeStruct((B, G), _f32),
        scratch_shapes=[pltpu.VMEM((1, H), _f32), pltpu.VMEM((H, H), _f32),
                        pltpu.VMEM((1, H), _f32), pltpu.VMEM((1, H), _f32),
                        pltpu.VMEM((B, G, H), _f32), pltpu.VMEM((G, B), _f32)],
    )(p['gene_emb'], alpha, cvec2,
      p['rec_W0'], row(p['rec_b0']), row(p['rec_g0']), row(p['rec_be0']),
      p['rec_W1'], row(p['rec_b1']), row(p['rec_g1']), row(p['rec_be1']),
      p['indv_w1'][:, :, 0], p['indv_b1'], x2,
      p['cg_W0'], row(p['cg_b0']), row(p['cg_g0']), row(p['cg_be0']),
      p['cg_W1'], row(p['cg_b1']), row(p['cg_g1']), row(p['cg_be1']),
      p['indv_w2'][0, :, 0].reshape(1, G), p['indv_w2'][0, :, 1:],
      p['indv_b2'][0].reshape(1, G))
    return final
